# baseline (speedup 1.0000x reference)
import functools

import jax
import jax.numpy as jnp
from jax import lax
from jax.experimental import pallas as pl
from jax.experimental.pallas import tpu as pltpu
from jax.experimental.pallas import tpu_sc as plsc

N_NODES = 10000
N_REL = 8
N_EDGES = 320000

NC = 2
NS = 16
NW = NC * NS
LANES = 16

ET = N_EDGES // NW
KPAD = 81920
NPAD = 10240
ROWS_PER_TILE = NPAD // NS

_MESH = dict(core_axis_name="c", subcore_axis_name="s")


def _wid():
    return lax.axis_index("s") * NC + lax.axis_index("c")


def _fill(ref, n, value, dtype):
    vec = jnp.full((LANES,), value, dtype)

    def body(i, _):
        ref[pl.ds(i * LANES, LANES)] = vec
        return 0

    lax.fori_loop(0, n // LANES, body, 0)


def _count_body(etype, src, dst, part_out, skey_out, gkey_out,
                tbuf, sbuf, dbuf, kbuf, gbuf, ones, zbuf, cnt_sh):
    wid = _wid()
    sid = lax.axis_index("s")
    cid = lax.axis_index("c")

    CB = 128
    NCHUNK = N_EDGES // CB
    NITER = -(-NCHUNK // NW)

    _fill(ones, CB, 1.0, jnp.float32)
    _fill(zbuf, KPAD // NS, 0.0, jnp.float32)
    pltpu.sync_copy(zbuf, cnt_sh.at[pl.ds(sid * (KPAD // NS), KPAD // NS)])
    plsc.subcore_barrier()

    def chunk(i, _):
        cidx = i * NW + wid

        @pl.when(cidx < NCHUNK)
        def _():
            base = cidx * CB
            pltpu.sync_copy(etype.at[pl.ds(base, CB)], tbuf)
            pltpu.sync_copy(src.at[pl.ds(base, CB)], sbuf)
            pltpu.sync_copy(dst.at[pl.ds(base, CB)], dbuf)

            def body(j, _):
                sl = pl.ds(j * LANES, LANES)
                t = tbuf[sl] * N_NODES
                kbuf[sl] = t + dbuf[sl]
                gbuf[sl] = t + sbuf[sl]
                return 0

            lax.fori_loop(0, CB // LANES, body, 0)
            pltpu.sync_copy(kbuf, skey_out.at[pl.ds(base, CB)])
            pltpu.sync_copy(gbuf, gkey_out.at[pl.ds(base, CB)])
            pltpu.sync_copy(ones, cnt_sh.at[kbuf], add=True)

        return 0

    lax.fori_loop(0, NITER, chunk, 0)

    plsc.subcore_barrier()
    off = sid * (KPAD // NS)
    pltpu.sync_copy(cnt_sh.at[pl.ds(off, KPAD // NS)],
                    part_out.at[cid, pl.ds(off, KPAD // NS)])


def _sc_count(etype, src, dst):
    mesh = plsc.VectorSubcoreMesh(**_MESH)
    f = pl.kernel(
        _count_body,
        out_type=(
            jax.ShapeDtypeStruct((NC, KPAD), jnp.float32),
            jax.ShapeDtypeStruct((N_EDGES,), jnp.int32),
            jax.ShapeDtypeStruct((N_EDGES,), jnp.int32),
        ),
        mesh=mesh,
        compiler_params=pltpu.CompilerParams(needs_layout_passes=False),
        scratch_types=[
            pltpu.VMEM((128,), jnp.int32),
            pltpu.VMEM((128,), jnp.int32),
            pltpu.VMEM((128,), jnp.int32),
            pltpu.VMEM((128,), jnp.int32),
            pltpu.VMEM((128,), jnp.int32),
            pltpu.VMEM((128,), jnp.float32),
            pltpu.VMEM((KPAD // NS,), jnp.float32),
            pltpu.VMEM_SHARED((KPAD,), jnp.float32),
        ],
    )
    return f(etype, src, dst)


def _scale_body(part, skey, c_out, abuf, bbuf, kbuf, cbuf, tab, cnt_sh):
    sid = lax.axis_index("s")
    wid = _wid()

    W = KPAD // NS
    off = sid * W
    pltpu.sync_copy(part.at[0, pl.ds(off, W)], abuf)
    pltpu.sync_copy(part.at[1, pl.ds(off, W)], bbuf)

    def body(i, _):
        sl = pl.ds(i * LANES, LANES)
        abuf[sl] = abuf[sl] + bbuf[sl]
        return 0

    lax.fori_loop(0, W // LANES, body, 0)
    pltpu.sync_copy(abuf, cnt_sh.at[pl.ds(off, W)])
    plsc.subcore_barrier()

    pltpu.sync_copy(cnt_sh, tab)

    CB = 2000
    one = jnp.full((LANES,), 1.0, jnp.float32)

    for k in range(ET // CB):
        base = wid * ET + k * CB
        pltpu.sync_copy(skey.at[pl.ds(base, CB)], kbuf)

        def body(i, _):
            sl = pl.ds(i * LANES, LANES)
            cnt = plsc.load_gather(tab, [kbuf[sl]])
            cbuf[sl] = one / jnp.maximum(cnt, one)
            return 0

        lax.fori_loop(0, CB // LANES, body, 0)
        pltpu.sync_copy(cbuf, c_out.at[pl.ds(base, CB)])


def _sc_scale(part, skey):
    mesh = plsc.VectorSubcoreMesh(**_MESH)
    W = KPAD // NS
    f = pl.kernel(
        _scale_body,
        out_type=jax.ShapeDtypeStruct((N_EDGES,), jnp.float32),
        mesh=mesh,
        compiler_params=pltpu.CompilerParams(needs_layout_passes=False),
        scratch_types=[
            pltpu.VMEM((W,), jnp.float32),
            pltpu.VMEM((W,), jnp.float32),
            pltpu.VMEM((2000,), jnp.int32),
            pltpu.VMEM((2000,), jnp.float32),
            pltpu.VMEM((KPAD,), jnp.float32),
            pltpu.VMEM_SHARED((KPAD,), jnp.float32),
        ],
    )
    return f(part, skey)


def _agg_body(D, xr, gkey, dst, c, part_out,
              gbuf, dbuf, cbuf, rows, sem,
              gbuf2, dbuf2, cbuf2, rows2, sem2, acc_sh):
    wid = _wid()
    sid = lax.axis_index("s")
    cid = lax.axis_index("c")
    DV = D // LANES
    B = 128
    NCHUNK = N_EDGES // B
    NITER = -(-NCHUNK // NW)

    def zbody(e, _):
        for j in range(DV):
            rows[e, pl.ds(j * LANES, LANES)] = jnp.zeros((LANES,), jnp.float32)
        return 0

    lax.fori_loop(0, B, zbody, 0)
    for z in range(ROWS_PER_TILE // B):
        pltpu.sync_copy(rows,
                        acc_sh.at[pl.ds(sid * ROWS_PER_TILE + z * B, B), :])
    plsc.subcore_barrier()

    bufs = ((gbuf, dbuf, cbuf, rows, sem), (gbuf2, dbuf2, cbuf2, rows2, sem2))

    def load_fire(j, bi):
        g, d, cc, r, sm = bufs[bi]
        cidx = j * NW + wid

        @pl.when(cidx < NCHUNK)
        def _():
            base = cidx * B
            pltpu.sync_copy(gkey.at[pl.ds(base, B)], g)
            pltpu.async_copy(xr.at[g], r, sm)
            pltpu.sync_copy(c.at[pl.ds(base, B)], cc)
            pltpu.sync_copy(dst.at[pl.ds(base, B)], d)

    def drain_process(j, bi):
        g, d, cc, r, sm = bufs[bi]
        cidx = j * NW + wid

        @pl.when(cidx < NCHUNK)
        def _():
            pltpu.make_async_copy(xr.at[g], r, sm).wait()

            def body(e, _):
                cv = plsc.load_gather(cc, [jnp.full((LANES,), e, jnp.int32)])
                for jj in range(DV):
                    sl = pl.ds(jj * LANES, LANES)
                    r[e, sl] = r[e, sl] * cv
                return 0

            lax.fori_loop(0, B, body, 0)
            pltpu.sync_copy(r, acc_sh.at[d], add=True)

    load_fire(0, 0)

    def pair(i, _):
        load_fire(2 * i + 1, 1)
        drain_process(2 * i, 0)
        load_fire(2 * i + 2, 0)
        drain_process(2 * i + 1, 1)
        return 0

    lax.fori_loop(0, (NITER + 1) // 2, pair, 0)

    plsc.subcore_barrier()
    roff = sid * ROWS_PER_TILE
    for z in range(ROWS_PER_TILE // B):
        pltpu.sync_copy(acc_sh.at[pl.ds(roff + z * B, B), :],
                        part_out.at[cid, pl.ds(roff + z * B, B), :])


def _sc_agg(xr, gkey, dst, c, D):
    B = 128
    mesh = plsc.VectorSubcoreMesh(**_MESH)
    f = pl.kernel(
        functools.partial(_agg_body, D),
        out_type=jax.ShapeDtypeStruct((NC, NPAD, D), jnp.float32),
        mesh=mesh,
        compiler_params=pltpu.CompilerParams(needs_layout_passes=False),
        scratch_types=[
            pltpu.VMEM((B,), jnp.int32),
            pltpu.VMEM((B,), jnp.int32),
            pltpu.VMEM((B,), jnp.float32),
            pltpu.VMEM((B, D), jnp.float32),
            pltpu.SemaphoreType.DMA,
            pltpu.VMEM((B,), jnp.int32),
            pltpu.VMEM((B,), jnp.int32),
            pltpu.VMEM((B,), jnp.float32),
            pltpu.VMEM((B, D), jnp.float32),
            pltpu.SemaphoreType.DMA,
            pltpu.VMEM_SHARED((NPAD, D), jnp.float32),
        ],
    )
    return f(xr, gkey, dst, c)


def _slot_body(xr, gkey, dst, c, part_out,
               kbuf, gbuf, sbuf, dbuf, cbuf, rows, srows, sem,
               kbuf2, gbuf2, sbuf2, dbuf2, cbuf2, rows2, srows2, sem2,
               acc_sh):
    wid = _wid()
    sid = lax.axis_index("s")
    cid = lax.axis_index("c")
    B = 64
    NCHUNK = N_EDGES // B
    NITER = -(-NCHUNK // NW)
    iota = lax.iota(jnp.int32, LANES)
    zvec = jnp.zeros((LANES,), jnp.float32)

    def zbody(e, _):
        for j in range(8):
            sl = pl.ds(j * LANES, LANES)
            rows[e, sl] = zvec
            srows[e, sl] = zvec
            srows2[e, sl] = zvec
        return 0

    lax.fori_loop(0, B, zbody, 0)
    for z in range(ROWS_PER_TILE // B):
        pltpu.sync_copy(rows,
                        acc_sh.at[pl.ds(sid * ROWS_PER_TILE + z * B, B), :])
    plsc.subcore_barrier()

    bufs = ((kbuf, gbuf, sbuf, dbuf, cbuf, rows, srows, sem),
            (kbuf2, gbuf2, sbuf2, dbuf2, cbuf2, rows2, srows2, sem2))

    def load_fire(j, bi):
        kb_, gb, sb, db, cb, r, _sr, sm = bufs[bi]
        cidx = j * NW + wid

        @pl.when(cidx < NCHUNK)
        def _():
            base = cidx * B
            pltpu.sync_copy(gkey.at[pl.ds(base, B)], kb_)

            def kb(j2, _):
                sl = pl.ds(j2 * LANES, LANES)
                k16 = kb_[sl]
                gb[sl] = lax.shift_right_logical(k16, 3)
                sb[sl] = (k16 & 7) * LANES
                return 0

            lax.fori_loop(0, B // LANES, kb, 0)
            pltpu.async_copy(xr.at[gb], r, sm)
            pltpu.sync_copy(c.at[pl.ds(base, B)], cb)
            pltpu.sync_copy(dst.at[pl.ds(base, B)], db)

    def drain_process(j, bi):
        _kb, gb, sb, db, cb, r, sr, sm = bufs[bi]
        cidx = j * NW + wid

        @pl.when(cidx < NCHUNK)
        def _():
            pltpu.make_async_copy(xr.at[gb], r, sm).wait()

            def body(e, _):
                ef = jnp.full((LANES,), e, jnp.int32)
                col = plsc.load_gather(sb, [ef]) + iota
                msg = plsc.load_gather(r, [ef, col])
                cv = plsc.load_gather(cb, [ef])
                plsc.store_scatter(sr, [ef, col], msg * cv)
                return 0

            lax.fori_loop(0, B, body, 0)
            pltpu.sync_copy(sr, acc_sh.at[db], add=True)

            def restore(e, _):
                ef = jnp.full((LANES,), e, jnp.int32)
                col = plsc.load_gather(sb, [ef]) + iota
                plsc.store_scatter(sr, [ef, col], zvec)
                return 0

            lax.fori_loop(0, B, restore, 0)

    load_fire(0, 0)

    def pair(i, _):
        load_fire(2 * i + 1, 1)
        drain_process(2 * i, 0)
        load_fire(2 * i + 2, 0)
        drain_process(2 * i + 1, 1)
        return 0

    lax.fori_loop(0, (NITER + 1) // 2, pair, 0)

    plsc.subcore_barrier()
    roff = sid * ROWS_PER_TILE
    for z in range(ROWS_PER_TILE // B):
        pltpu.sync_copy(acc_sh.at[pl.ds(roff + z * B, B), :],
                        part_out.at[cid, pl.ds(roff + z * B, B), :])


def _sc_slot_agg(xr, gkey, dst, c):
    B = 64
    mesh = plsc.VectorSubcoreMesh(**_MESH)
    f = pl.kernel(
        _slot_body,
        out_type=jax.ShapeDtypeStruct((NC, NPAD, 128), jnp.float32),
        mesh=mesh,
        compiler_params=pltpu.CompilerParams(needs_layout_passes=False),
        scratch_types=[
            pltpu.VMEM((B,), jnp.int32),
            pltpu.VMEM((B,), jnp.int32),
            pltpu.VMEM((B,), jnp.int32),
            pltpu.VMEM((B,), jnp.int32),
            pltpu.VMEM((B,), jnp.float32),
            pltpu.VMEM((B, 128), jnp.float32),
            pltpu.VMEM((B, 128), jnp.float32),
            pltpu.SemaphoreType.DMA,
            pltpu.VMEM((B,), jnp.int32),
            pltpu.VMEM((B,), jnp.int32),
            pltpu.VMEM((B,), jnp.int32),
            pltpu.VMEM((B,), jnp.int32),
            pltpu.VMEM((B,), jnp.float32),
            pltpu.VMEM((B, 128), jnp.float32),
            pltpu.VMEM((B, 128), jnp.float32),
            pltpu.SemaphoreType.DMA,
            pltpu.VMEM_SHARED((NPAD, 128), jnp.float32),
        ],
    )
    return f(xr, gkey, dst, c)


def _mm_body(x_ref, w_ref, b_ref, xr_ref, root_ref):
    r = pl.program_id(1)
    acc = jnp.dot(x_ref[...], w_ref[0], preferred_element_type=jnp.float32)

    @pl.when(r < N_REL)
    def _():
        xr_ref[...] = acc

    @pl.when(r == N_REL)
    def _():
        root_ref[...] = acc + b_ref[0]


def _tc_matmul(x, w_all, b, dout, bn):
    n = x.shape[0]
    nb = n // bn
    return pl.pallas_call(
        _mm_body,
        grid=(nb, N_REL + 1),
        in_specs=[
            pl.BlockSpec((bn, x.shape[1]), lambda i, r: (i, 0)),
            pl.BlockSpec((1, x.shape[1], dout), lambda i, r: (r, 0, 0)),
            pl.BlockSpec((1, dout), lambda i, r: (0, 0)),
        ],
        out_specs=[
            pl.BlockSpec((bn, dout),
                         lambda i, r: (jnp.minimum(r, N_REL - 1) * (n // bn) + i, 0)),
            pl.BlockSpec((bn, dout), lambda i, r: (i, 0)),
        ],
        out_shape=[
            jax.ShapeDtypeStruct((N_REL * n, dout), jnp.float32),
            jax.ShapeDtypeStruct((n, dout), jnp.float32),
        ],
    )(x, w_all, b.reshape(1, dout))


def _comb_body(relu, slots, root_ref, p0_ref, p1_ref, o_ref):
    p = p0_ref[0] + p1_ref[0]
    if slots:
        bn = p.shape[0]
        p = p.reshape(bn, 8, p.shape[1] // 8).sum(axis=1)
    v = root_ref[...] + p
    if relu:
        v = jnp.maximum(v, 0.0)
    o_ref[...] = v


def _tc_combine(root, part, relu, bn, slots=False):
    n, d = root.shape
    pd = part.shape[2]
    return pl.pallas_call(
        functools.partial(_comb_body, relu, slots),
        grid=(n // bn,),
        in_specs=[
            pl.BlockSpec((bn, d), lambda i: (i, 0)),
            pl.BlockSpec((1, bn, pd), lambda i: (0, i, 0)),
            pl.BlockSpec((1, bn, pd), lambda i: (1, i, 0)),
        ],
        out_specs=pl.BlockSpec((bn, d), lambda i: (i, 0)),
        out_shape=jax.ShapeDtypeStruct((n, d), jnp.float32),
    )(root, part, part)


def kernel(x, edge_index, edge_type, W_rel1, W_root1, b1, W_rel2, W_root2, b2):
    src = edge_index[0]
    dst = edge_index[1]

    part_cnt, skey, gkey = _sc_count(edge_type, src, dst)
    c = _sc_scale(part_cnt, skey)

    w_all1 = jnp.concatenate([W_rel1, W_root1[None]], axis=0)
    xr1, root1 = _tc_matmul(x, w_all1, b1, 128, 1000)
    part1 = _sc_agg(xr1, gkey, dst, c, D=128)
    h = _tc_combine(root1, part1, relu=True, bn=1000)

    w_all2 = jnp.concatenate([W_rel2, W_root2[None]], axis=0)
    xr2, root2 = _tc_matmul(h, w_all2, b2, 16, 1000)
    part2 = _sc_slot_agg(xr2.reshape(N_NODES, 128), gkey, dst, c)
    logits = _tc_combine(root2, part2, relu=False, bn=1000, slots=True)
    return logits


# speedup vs baseline: 1.2526x; 1.2526x over previous
import functools

import jax
import jax.numpy as jnp
from jax import lax
from jax.experimental import pallas as pl
from jax.experimental.pallas import tpu as pltpu
from jax.experimental.pallas import tpu_sc as plsc

N_NODES = 10000
N_REL = 8
N_EDGES = 320000

NC = 2
NS = 16
NW = NC * NS
LANES = 16

ET = N_EDGES // NW
KPAD = 81920
NPAD = 10240
ROWS_PER_TILE = NPAD // NS

_MESH = dict(core_axis_name="c", subcore_axis_name="s")


def _wid():
    return lax.axis_index("s") * NC + lax.axis_index("c")


def _fill(ref, n, value, dtype):
    vec = jnp.full((LANES,), value, dtype)

    def body(i, _):
        ref[pl.ds(i * LANES, LANES)] = vec
        return 0

    lax.fori_loop(0, n // LANES, body, 0)


def _count_body(etype, src, dst, part_out, skey_out, gkey_out,
                tbuf, sbuf, dbuf, kbuf, gbuf, ones, zbuf, cnt_sh):
    wid = _wid()
    sid = lax.axis_index("s")
    cid = lax.axis_index("c")

    CB = 128
    NCHUNK = N_EDGES // CB
    NITER = -(-NCHUNK // NW)

    _fill(ones, CB, 1.0, jnp.float32)
    _fill(zbuf, KPAD // NS, 0.0, jnp.float32)
    pltpu.sync_copy(zbuf, cnt_sh.at[pl.ds(sid * (KPAD // NS), KPAD // NS)])
    plsc.subcore_barrier()

    def chunk(i, _):
        cidx = i * NW + wid

        @pl.when(cidx < NCHUNK)
        def _():
            base = cidx * CB
            pltpu.sync_copy(etype.at[pl.ds(base, CB)], tbuf)
            pltpu.sync_copy(src.at[pl.ds(base, CB)], sbuf)
            pltpu.sync_copy(dst.at[pl.ds(base, CB)], dbuf)

            def body(j, _):
                sl = pl.ds(j * LANES, LANES)
                t = tbuf[sl] * N_NODES
                kbuf[sl] = t + dbuf[sl]
                gbuf[sl] = t + sbuf[sl]
                return 0

            lax.fori_loop(0, CB // LANES, body, 0)
            pltpu.sync_copy(kbuf, skey_out.at[pl.ds(base, CB)])
            pltpu.sync_copy(gbuf, gkey_out.at[pl.ds(base, CB)])
            pltpu.sync_copy(ones, cnt_sh.at[kbuf], add=True)

        return 0

    lax.fori_loop(0, NITER, chunk, 0)

    plsc.subcore_barrier()
    off = sid * (KPAD // NS)
    pltpu.sync_copy(cnt_sh.at[pl.ds(off, KPAD // NS)],
                    part_out.at[cid, pl.ds(off, KPAD // NS)])


def _sc_count(etype, src, dst):
    mesh = plsc.VectorSubcoreMesh(**_MESH)
    f = pl.kernel(
        _count_body,
        out_type=(
            jax.ShapeDtypeStruct((NC, KPAD), jnp.float32),
            jax.ShapeDtypeStruct((N_EDGES,), jnp.int32),
            jax.ShapeDtypeStruct((N_EDGES,), jnp.int32),
        ),
        mesh=mesh,
        compiler_params=pltpu.CompilerParams(needs_layout_passes=False),
        scratch_types=[
            pltpu.VMEM((128,), jnp.int32),
            pltpu.VMEM((128,), jnp.int32),
            pltpu.VMEM((128,), jnp.int32),
            pltpu.VMEM((128,), jnp.int32),
            pltpu.VMEM((128,), jnp.int32),
            pltpu.VMEM((128,), jnp.float32),
            pltpu.VMEM((KPAD // NS,), jnp.float32),
            pltpu.VMEM_SHARED((KPAD,), jnp.float32),
        ],
    )
    return f(etype, src, dst)


def _scale_body(part, skey, c_out, abuf, bbuf, kbuf, cbuf, tab, cnt_sh):
    sid = lax.axis_index("s")
    wid = _wid()

    W = KPAD // NS
    off = sid * W
    pltpu.sync_copy(part.at[0, pl.ds(off, W)], abuf)
    pltpu.sync_copy(part.at[1, pl.ds(off, W)], bbuf)

    def body(i, _):
        sl = pl.ds(i * LANES, LANES)
        abuf[sl] = abuf[sl] + bbuf[sl]
        return 0

    lax.fori_loop(0, W // LANES, body, 0)
    pltpu.sync_copy(abuf, cnt_sh.at[pl.ds(off, W)])
    plsc.subcore_barrier()

    pltpu.sync_copy(cnt_sh, tab)

    CB = 2000
    one = jnp.full((LANES,), 1.0, jnp.float32)

    for k in range(ET // CB):
        base = wid * ET + k * CB
        pltpu.sync_copy(skey.at[pl.ds(base, CB)], kbuf)

        def body(i, _):
            sl = pl.ds(i * LANES, LANES)
            cnt = plsc.load_gather(tab, [kbuf[sl]])
            cbuf[sl] = one / jnp.maximum(cnt, one)
            return 0

        lax.fori_loop(0, CB // LANES, body, 0)
        pltpu.sync_copy(cbuf, c_out.at[pl.ds(base, CB)])


def _sc_scale(part, skey):
    mesh = plsc.VectorSubcoreMesh(**_MESH)
    W = KPAD // NS
    f = pl.kernel(
        _scale_body,
        out_type=jax.ShapeDtypeStruct((N_EDGES,), jnp.float32),
        mesh=mesh,
        compiler_params=pltpu.CompilerParams(needs_layout_passes=False),
        scratch_types=[
            pltpu.VMEM((W,), jnp.float32),
            pltpu.VMEM((W,), jnp.float32),
            pltpu.VMEM((2000,), jnp.int32),
            pltpu.VMEM((2000,), jnp.float32),
            pltpu.VMEM((KPAD,), jnp.float32),
            pltpu.VMEM_SHARED((KPAD,), jnp.float32),
        ],
    )
    return f(part, skey)


def _agg_body(D, xr, gkey, dst, c, part_out,
              gbuf, dbuf, cbuf, rows, sem,
              gbuf2, dbuf2, cbuf2, rows2, sem2, acc_sh):
    wid = _wid()
    sid = lax.axis_index("s")
    cid = lax.axis_index("c")
    DV = D // LANES
    B = 128
    NCHUNK = N_EDGES // B
    NITER = -(-NCHUNK // NW)

    def zbody(e, _):
        for j in range(DV):
            rows[e, pl.ds(j * LANES, LANES)] = jnp.zeros((LANES,), jnp.float32)
        return 0

    lax.fori_loop(0, B, zbody, 0)
    for z in range(ROWS_PER_TILE // B):
        pltpu.sync_copy(rows,
                        acc_sh.at[pl.ds(sid * ROWS_PER_TILE + z * B, B), :])
    plsc.subcore_barrier()

    bufs = ((gbuf, dbuf, cbuf, rows, sem), (gbuf2, dbuf2, cbuf2, rows2, sem2))

    def load_fire(j, bi):
        g, d, cc, r, sm = bufs[bi]
        cidx = j * NW + wid

        @pl.when(cidx < NCHUNK)
        def _():
            base = cidx * B
            pltpu.sync_copy(gkey.at[pl.ds(base, B)], g)
            pltpu.async_copy(xr.at[g], r, sm)
            pltpu.sync_copy(c.at[pl.ds(base, B)], cc)
            pltpu.sync_copy(dst.at[pl.ds(base, B)], d)

    def drain_process(j, bi):
        g, d, cc, r, sm = bufs[bi]
        cidx = j * NW + wid

        @pl.when(cidx < NCHUNK)
        def _():
            pltpu.make_async_copy(xr.at[g], r, sm).wait()

            def body(e, _):
                cv = plsc.load_gather(cc, [jnp.full((LANES,), e, jnp.int32)])
                for jj in range(DV):
                    sl = pl.ds(jj * LANES, LANES)
                    r[e, sl] = r[e, sl] * cv
                return 0

            lax.fori_loop(0, B, body, 0)
            pltpu.sync_copy(r, acc_sh.at[d], add=True)

    load_fire(0, 0)

    def pair(i, _):
        load_fire(2 * i + 1, 1)
        drain_process(2 * i, 0)
        load_fire(2 * i + 2, 0)
        drain_process(2 * i + 1, 1)
        return 0

    lax.fori_loop(0, (NITER + 1) // 2, pair, 0)

    plsc.subcore_barrier()
    roff = sid * ROWS_PER_TILE
    for z in range(ROWS_PER_TILE // B):
        pltpu.sync_copy(acc_sh.at[pl.ds(roff + z * B, B), :],
                        part_out.at[cid, pl.ds(roff + z * B, B), :])


def _sc_agg(xr, gkey, dst, c, D):
    B = 128
    mesh = plsc.VectorSubcoreMesh(**_MESH)
    f = pl.kernel(
        functools.partial(_agg_body, D),
        out_type=jax.ShapeDtypeStruct((NC, NPAD, D), jnp.float32),
        mesh=mesh,
        compiler_params=pltpu.CompilerParams(needs_layout_passes=False),
        scratch_types=[
            pltpu.VMEM((B,), jnp.int32),
            pltpu.VMEM((B,), jnp.int32),
            pltpu.VMEM((B,), jnp.float32),
            pltpu.VMEM((B, D), jnp.float32),
            pltpu.SemaphoreType.DMA,
            pltpu.VMEM((B,), jnp.int32),
            pltpu.VMEM((B,), jnp.int32),
            pltpu.VMEM((B,), jnp.float32),
            pltpu.VMEM((B, D), jnp.float32),
            pltpu.SemaphoreType.DMA,
            pltpu.VMEM_SHARED((NPAD, D), jnp.float32),
        ],
    )
    return f(xr, gkey, dst, c)


def _slot_body(xr, gkey, dst, c, part_out,
               kbuf, gbuf, sbuf, dbuf, cbuf, rows, sem,
               kbuf2, gbuf2, sbuf2, dbuf2, cbuf2, rows2, sem2,
               acc_sh):
    wid = _wid()
    sid = lax.axis_index("s")
    cid = lax.axis_index("c")
    B = 128
    NCHUNK = N_EDGES // B
    NITER = -(-NCHUNK // NW)
    iota = lax.iota(jnp.int32, LANES)
    zvec = jnp.zeros((LANES,), jnp.float32)

    def zbody(e, _):
        for j in range(8):
            rows[e, pl.ds(j * LANES, LANES)] = zvec
        return 0

    lax.fori_loop(0, B, zbody, 0)
    for z in range(ROWS_PER_TILE // B):
        pltpu.sync_copy(rows,
                        acc_sh.at[pl.ds(sid * ROWS_PER_TILE + z * B, B), :])
    plsc.subcore_barrier()

    bufs = ((kbuf, gbuf, sbuf, dbuf, cbuf, rows, sem),
            (kbuf2, gbuf2, sbuf2, dbuf2, cbuf2, rows2, sem2))

    def load_fire(j, bi):
        kb_, gb, sb, db, cb, r, sm = bufs[bi]
        cidx = j * NW + wid

        @pl.when(cidx < NCHUNK)
        def _():
            base = cidx * B
            pltpu.sync_copy(gkey.at[pl.ds(base, B)], kb_)

            def kb(j2, _):
                sl = pl.ds(j2 * LANES, LANES)
                k16 = kb_[sl]
                gb[sl] = lax.shift_right_logical(k16, 3)
                sb[sl] = (k16 & 7) * LANES
                return 0

            lax.fori_loop(0, B // LANES, kb, 0)
            pltpu.async_copy(xr.at[gb], r, sm)
            pltpu.sync_copy(c.at[pl.ds(base, B)], cb)
            pltpu.sync_copy(dst.at[pl.ds(base, B)], db)

    def drain_process(j, bi):
        _kb, gb, sb, db, cb, r, sm = bufs[bi]
        cidx = j * NW + wid

        @pl.when(cidx < NCHUNK)
        def _():
            pltpu.make_async_copy(xr.at[gb], r, sm).wait()

            def body(e, _):
                ef = jnp.full((LANES,), e, jnp.int32)
                col = plsc.load_gather(sb, [ef]) + iota
                msg = plsc.load_gather(r, [ef, col])
                cv = plsc.load_gather(cb, [ef])
                for j2 in range(8):
                    r[e, pl.ds(j2 * LANES, LANES)] = zvec
                plsc.store_scatter(r, [ef, col], msg * cv)
                return 0

            lax.fori_loop(0, B, body, 0)
            pltpu.sync_copy(r, acc_sh.at[db], add=True)

    load_fire(0, 0)

    def pair(i, _):
        load_fire(2 * i + 1, 1)
        drain_process(2 * i, 0)
        load_fire(2 * i + 2, 0)
        drain_process(2 * i + 1, 1)
        return 0

    lax.fori_loop(0, (NITER + 1) // 2, pair, 0)

    plsc.subcore_barrier()

    roff = sid * ROWS_PER_TILE
    for z in range(ROWS_PER_TILE // B):
        pltpu.sync_copy(acc_sh.at[pl.ds(roff + z * B, B), :],
                        part_out.at[cid, pl.ds(roff + z * B, B), :])


def _sc_slot_agg(xr, gkey, dst, c):
    B = 128
    mesh = plsc.VectorSubcoreMesh(**_MESH)
    f = pl.kernel(
        _slot_body,
        out_type=jax.ShapeDtypeStruct((NC, NPAD, 128), jnp.float32),
        mesh=mesh,
        compiler_params=pltpu.CompilerParams(needs_layout_passes=False),
        scratch_types=[
            pltpu.VMEM((B,), jnp.int32),
            pltpu.VMEM((B,), jnp.int32),
            pltpu.VMEM((B,), jnp.int32),
            pltpu.VMEM((B,), jnp.int32),
            pltpu.VMEM((B,), jnp.float32),
            pltpu.VMEM((B, 128), jnp.float32),
            pltpu.SemaphoreType.DMA,
            pltpu.VMEM((B,), jnp.int32),
            pltpu.VMEM((B,), jnp.int32),
            pltpu.VMEM((B,), jnp.int32),
            pltpu.VMEM((B,), jnp.int32),
            pltpu.VMEM((B,), jnp.float32),
            pltpu.VMEM((B, 128), jnp.float32),
            pltpu.SemaphoreType.DMA,
            pltpu.VMEM_SHARED((NPAD, 128), jnp.float32),
        ],
    )
    return f(xr, gkey, dst, c)


def _mm_body(x_ref, w_ref, b_ref, xr_ref, root_ref):
    r = pl.program_id(1)
    acc = jnp.dot(x_ref[...], w_ref[0], preferred_element_type=jnp.float32)

    @pl.when(r < N_REL)
    def _():
        xr_ref[...] = acc

    @pl.when(r == N_REL)
    def _():
        root_ref[...] = acc + b_ref[0]


def _tc_matmul(x, w_all, b, dout, bn):
    n = x.shape[0]
    nb = n // bn
    return pl.pallas_call(
        _mm_body,
        grid=(nb, N_REL + 1),
        in_specs=[
            pl.BlockSpec((bn, x.shape[1]), lambda i, r: (i, 0)),
            pl.BlockSpec((1, x.shape[1], dout), lambda i, r: (r, 0, 0)),
            pl.BlockSpec((1, dout), lambda i, r: (0, 0)),
        ],
        out_specs=[
            pl.BlockSpec((bn, dout),
                         lambda i, r: (jnp.minimum(r, N_REL - 1) * (n // bn) + i, 0)),
            pl.BlockSpec((bn, dout), lambda i, r: (i, 0)),
        ],
        out_shape=[
            jax.ShapeDtypeStruct((N_REL * n, dout), jnp.float32),
            jax.ShapeDtypeStruct((n, dout), jnp.float32),
        ],
    )(x, w_all, b.reshape(1, dout))


def _comb_body(relu, slots, root_ref, p0_ref, p1_ref, o_ref):
    p = p0_ref[0] + p1_ref[0]
    if slots:
        bn = p.shape[0]
        p = p.reshape(bn, 8, p.shape[1] // 8).sum(axis=1)
    v = root_ref[...] + p
    if relu:
        v = jnp.maximum(v, 0.0)
    o_ref[...] = v


def _tc_combine(root, part, relu, bn, slots=False):
    n, d = root.shape
    pd = part.shape[2]
    return pl.pallas_call(
        functools.partial(_comb_body, relu, slots),
        grid=(n // bn,),
        in_specs=[
            pl.BlockSpec((bn, d), lambda i: (i, 0)),
            pl.BlockSpec((1, bn, pd), lambda i: (0, i, 0)),
            pl.BlockSpec((1, bn, pd), lambda i: (1, i, 0)),
        ],
        out_specs=pl.BlockSpec((bn, d), lambda i: (i, 0)),
        out_shape=jax.ShapeDtypeStruct((n, d), jnp.float32),
    )(root, part, part)


def kernel(x, edge_index, edge_type, W_rel1, W_root1, b1, W_rel2, W_root2, b2):
    src = edge_index[0]
    dst = edge_index[1]

    part_cnt, skey, gkey = _sc_count(edge_type, src, dst)
    c = _sc_scale(part_cnt, skey)

    w_all1 = jnp.concatenate([W_rel1, W_root1[None]], axis=0)
    xr1, root1 = _tc_matmul(x, w_all1, b1, 128, 1000)
    part1 = _sc_agg(xr1, gkey, dst, c, D=128)
    h = _tc_combine(root1, part1, relu=True, bn=1000)

    w_all2 = jnp.concatenate([W_rel2, W_root2[None]], axis=0)
    xr2, root2 = _tc_matmul(h, w_all2, b2, 16, 1000)
    part2 = _sc_slot_agg(xr2.reshape(N_NODES, 128), gkey, dst, c)
    logits = _tc_combine(root2, part2, relu=False, bn=1000, slots=True)
    return logits


# speedup vs baseline: 1.2527x; 1.0001x over previous
import functools

import jax
import jax.numpy as jnp
from jax import lax
from jax.experimental import pallas as pl
from jax.experimental.pallas import tpu as pltpu
from jax.experimental.pallas import tpu_sc as plsc

N_NODES = 10000
N_REL = 8
N_EDGES = 320000
CH1 = 128

NC = 2
NS = 16
NW = NC * NS
LANES = 16

ET = N_EDGES // NW
KPAD = 81920
NPAD = 10240
ROWS_PER_TILE = NPAD // NS

_MESH = dict(core_axis_name="c", subcore_axis_name="s")


def _wid():
    return lax.axis_index("s") * NC + lax.axis_index("c")


def _fill(ref, n, value, dtype):
    vec = jnp.full((LANES,), value, dtype)

    def body(i, _):
        ref[pl.ds(i * LANES, LANES)] = vec
        return 0

    lax.fori_loop(0, n // LANES, body, 0)


def _count_body(etype, src, dst, part_out, skey_out, gkey_out,
                tbuf, sbuf, dbuf, kbuf, gbuf, ones, zbuf, cnt_sh):
    wid = _wid()
    sid = lax.axis_index("s")
    cid = lax.axis_index("c")

    CB = 128
    NCHUNK = N_EDGES // CB
    NITER = -(-NCHUNK // NW)

    _fill(ones, CB, 1.0, jnp.float32)
    _fill(zbuf, KPAD // NS, 0.0, jnp.float32)
    pltpu.sync_copy(zbuf, cnt_sh.at[pl.ds(sid * (KPAD // NS), KPAD // NS)])
    plsc.subcore_barrier()

    def chunk(i, _):
        cidx = i * NW + wid

        @pl.when(cidx < NCHUNK)
        def _():
            base = cidx * CB
            pltpu.sync_copy(etype.at[pl.ds(base, CB)], tbuf)
            pltpu.sync_copy(src.at[pl.ds(base, CB)], sbuf)
            pltpu.sync_copy(dst.at[pl.ds(base, CB)], dbuf)

            def body(j, _):
                sl = pl.ds(j * LANES, LANES)
                t = tbuf[sl] * N_NODES
                kbuf[sl] = t + dbuf[sl]
                gbuf[sl] = t + sbuf[sl]
                return 0

            lax.fori_loop(0, CB // LANES, body, 0)
            pltpu.sync_copy(kbuf, skey_out.at[pl.ds(base, CB)])
            pltpu.sync_copy(gbuf, gkey_out.at[pl.ds(base, CB)])
            pltpu.sync_copy(ones, cnt_sh.at[kbuf], add=True)

        return 0

    lax.fori_loop(0, NITER, chunk, 0)

    plsc.subcore_barrier()
    off = sid * (KPAD // NS)
    pltpu.sync_copy(cnt_sh.at[pl.ds(off, KPAD // NS)],
                    part_out.at[cid, pl.ds(off, KPAD // NS)])


def _sc_count(etype, src, dst):
    mesh = plsc.VectorSubcoreMesh(**_MESH)
    f = pl.kernel(
        _count_body,
        out_type=(
            jax.ShapeDtypeStruct((NC, KPAD), jnp.float32),
            jax.ShapeDtypeStruct((N_EDGES,), jnp.int32),
            jax.ShapeDtypeStruct((N_EDGES,), jnp.int32),
        ),
        mesh=mesh,
        compiler_params=pltpu.CompilerParams(needs_layout_passes=False),
        scratch_types=[
            pltpu.VMEM((128,), jnp.int32),
            pltpu.VMEM((128,), jnp.int32),
            pltpu.VMEM((128,), jnp.int32),
            pltpu.VMEM((128,), jnp.int32),
            pltpu.VMEM((128,), jnp.int32),
            pltpu.VMEM((128,), jnp.float32),
            pltpu.VMEM((KPAD // NS,), jnp.float32),
            pltpu.VMEM_SHARED((KPAD,), jnp.float32),
        ],
    )
    return f(etype, src, dst)


def _scale_body(part, skey, c_out, abuf, bbuf, kbuf, cbuf, tab, cnt_sh):
    sid = lax.axis_index("s")
    wid = _wid()

    W = KPAD // NS
    off = sid * W
    pltpu.sync_copy(part.at[0, pl.ds(off, W)], abuf)
    pltpu.sync_copy(part.at[1, pl.ds(off, W)], bbuf)

    def body(i, _):
        sl = pl.ds(i * LANES, LANES)
        abuf[sl] = abuf[sl] + bbuf[sl]
        return 0

    lax.fori_loop(0, W // LANES, body, 0)
    pltpu.sync_copy(abuf, cnt_sh.at[pl.ds(off, W)])
    plsc.subcore_barrier()

    pltpu.sync_copy(cnt_sh, tab)

    CB = 2000
    one = jnp.full((LANES,), 1.0, jnp.float32)

    for k in range(ET // CB):
        base = wid * ET + k * CB
        pltpu.sync_copy(skey.at[pl.ds(base, CB)], kbuf)

        def body(i, _):
            sl = pl.ds(i * LANES, LANES)
            cnt = plsc.load_gather(tab, [kbuf[sl]])
            cbuf[sl] = one / jnp.maximum(cnt, one)
            return 0

        lax.fori_loop(0, CB // LANES, body, 0)
        pltpu.sync_copy(cbuf, c_out.at[pl.ds(base, CB)])


def _sc_scale(part, skey):
    mesh = plsc.VectorSubcoreMesh(**_MESH)
    W = KPAD // NS
    f = pl.kernel(
        _scale_body,
        out_type=jax.ShapeDtypeStruct((N_EDGES,), jnp.float32),
        mesh=mesh,
        compiler_params=pltpu.CompilerParams(needs_layout_passes=False),
        scratch_types=[
            pltpu.VMEM((W,), jnp.float32),
            pltpu.VMEM((W,), jnp.float32),
            pltpu.VMEM((2000,), jnp.int32),
            pltpu.VMEM((2000,), jnp.float32),
            pltpu.VMEM((KPAD,), jnp.float32),
            pltpu.VMEM_SHARED((KPAD,), jnp.float32),
        ],
    )
    return f(part, skey)


def _agg_body(D, xr, gkey, dst, c, part_out,
              gbuf, dbuf, cbuf, rows, sem,
              gbuf2, dbuf2, cbuf2, rows2, sem2, acc_sh):
    wid = _wid()
    sid = lax.axis_index("s")
    cid = lax.axis_index("c")
    DV = D // LANES
    B = 128
    NCHUNK = N_EDGES // B
    NITER = -(-NCHUNK // NW)

    def zbody(e, _):
        for j in range(DV):
            rows[e, pl.ds(j * LANES, LANES)] = jnp.zeros((LANES,), jnp.float32)
        return 0

    lax.fori_loop(0, B, zbody, 0)
    for z in range(ROWS_PER_TILE // B):
        pltpu.sync_copy(rows,
                        acc_sh.at[pl.ds(sid * ROWS_PER_TILE + z * B, B), :])
    plsc.subcore_barrier()

    bufs = ((gbuf, dbuf, cbuf, rows, sem), (gbuf2, dbuf2, cbuf2, rows2, sem2))

    def load_fire(j, bi):
        g, d, cc, r, sm = bufs[bi]
        cidx = j * NW + wid

        @pl.when(cidx < NCHUNK)
        def _():
            base = cidx * B
            pltpu.sync_copy(gkey.at[pl.ds(base, B)], g)
            pltpu.async_copy(xr.at[g], r, sm)
            pltpu.sync_copy(c.at[pl.ds(base, B)], cc)
            pltpu.sync_copy(dst.at[pl.ds(base, B)], d)

    def drain_process(j, bi):
        g, d, cc, r, sm = bufs[bi]
        cidx = j * NW + wid

        @pl.when(cidx < NCHUNK)
        def _():
            pltpu.make_async_copy(xr.at[g], r, sm).wait()

            def body(e, _):
                cv = plsc.load_gather(cc, [jnp.full((LANES,), e, jnp.int32)])
                for jj in range(DV):
                    sl = pl.ds(jj * LANES, LANES)
                    r[e, sl] = r[e, sl] * cv
                return 0

            lax.fori_loop(0, B, body, 0)
            pltpu.sync_copy(r, acc_sh.at[d], add=True)

    load_fire(0, 0)

    def pair(i, _):
        load_fire(2 * i + 1, 1)
        drain_process(2 * i, 0)
        load_fire(2 * i + 2, 0)
        drain_process(2 * i + 1, 1)
        return 0

    lax.fori_loop(0, (NITER + 1) // 2, pair, 0)

    plsc.subcore_barrier()
    roff = sid * ROWS_PER_TILE
    for z in range(ROWS_PER_TILE // B):
        pltpu.sync_copy(acc_sh.at[pl.ds(roff + z * B, B), :],
                        part_out.at[cid, pl.ds(roff + z * B, B), :])


def _sc_agg(xr, gkey, dst, c, D):
    B = 128
    mesh = plsc.VectorSubcoreMesh(**_MESH)
    f = pl.kernel(
        functools.partial(_agg_body, D),
        out_type=jax.ShapeDtypeStruct((NC, NPAD, D), jnp.float32),
        mesh=mesh,
        compiler_params=pltpu.CompilerParams(needs_layout_passes=False),
        scratch_types=[
            pltpu.VMEM((B,), jnp.int32),
            pltpu.VMEM((B,), jnp.int32),
            pltpu.VMEM((B,), jnp.float32),
            pltpu.VMEM((B, D), jnp.float32),
            pltpu.SemaphoreType.DMA,
            pltpu.VMEM((B,), jnp.int32),
            pltpu.VMEM((B,), jnp.int32),
            pltpu.VMEM((B,), jnp.float32),
            pltpu.VMEM((B, D), jnp.float32),
            pltpu.SemaphoreType.DMA,
            pltpu.VMEM_SHARED((NPAD, D), jnp.float32),
        ],
    )
    return f(xr, gkey, dst, c)


def _slot_body(xr, esrc, etype, dst, c, part_out,
               kbuf, gbuf, sbuf, dbuf, cbuf, rows, sem,
               kbuf2, gbuf2, sbuf2, dbuf2, cbuf2, rows2, sem2,
               acc_sh):
    wid = _wid()
    sid = lax.axis_index("s")
    cid = lax.axis_index("c")
    B = 128
    NCHUNK = N_EDGES // B
    NITER = -(-NCHUNK // NW)
    iota = lax.iota(jnp.int32, LANES)
    zvec = jnp.zeros((LANES,), jnp.float32)

    def zbody(e, _):
        for j in range(8):
            rows[e, pl.ds(j * LANES, LANES)] = zvec
        return 0

    lax.fori_loop(0, B, zbody, 0)
    for z in range(ROWS_PER_TILE // B):
        pltpu.sync_copy(rows,
                        acc_sh.at[pl.ds(sid * ROWS_PER_TILE + z * B, B), :])
    plsc.subcore_barrier()

    bufs = ((kbuf, gbuf, sbuf, dbuf, cbuf, rows, sem),
            (kbuf2, gbuf2, sbuf2, dbuf2, cbuf2, rows2, sem2))

    def load_fire(j, bi):
        kb_, gb, sb, db, cb, r, sm = bufs[bi]
        cidx = j * NW + wid

        @pl.when(cidx < NCHUNK)
        def _():
            base = cidx * B
            pltpu.sync_copy(esrc.at[pl.ds(base, B)], gb)
            pltpu.async_copy(xr.at[gb], r, sm)
            pltpu.sync_copy(etype.at[pl.ds(base, B)], kb_)

            def kb(j2, _):
                sl = pl.ds(j2 * LANES, LANES)
                sb[sl] = kb_[sl] * LANES
                return 0

            lax.fori_loop(0, B // LANES, kb, 0)
            pltpu.sync_copy(c.at[pl.ds(base, B)], cb)
            pltpu.sync_copy(dst.at[pl.ds(base, B)], db)

    def drain_process(j, bi):
        _kb, gb, sb, db, cb, r, sm = bufs[bi]
        cidx = j * NW + wid

        @pl.when(cidx < NCHUNK)
        def _():
            pltpu.make_async_copy(xr.at[gb], r, sm).wait()

            def body(e, _):
                ef = jnp.full((LANES,), e, jnp.int32)
                col = plsc.load_gather(sb, [ef]) + iota
                msg = plsc.load_gather(r, [ef, col])
                cv = plsc.load_gather(cb, [ef])
                for j2 in range(8):
                    r[e, pl.ds(j2 * LANES, LANES)] = zvec
                plsc.store_scatter(r, [ef, col], msg * cv)
                return 0

            lax.fori_loop(0, B, body, 0)
            pltpu.sync_copy(r, acc_sh.at[db], add=True)

    load_fire(0, 0)

    def pair(i, _):
        load_fire(2 * i + 1, 1)
        drain_process(2 * i, 0)
        load_fire(2 * i + 2, 0)
        drain_process(2 * i + 1, 1)
        return 0

    lax.fori_loop(0, (NITER + 1) // 2, pair, 0)

    plsc.subcore_barrier()

    roff = sid * ROWS_PER_TILE
    for z in range(ROWS_PER_TILE // B):
        pltpu.sync_copy(acc_sh.at[pl.ds(roff + z * B, B), :],
                        part_out.at[cid, pl.ds(roff + z * B, B), :])


def _sc_slot_agg(xr, esrc, etype, dst, c):
    B = 128
    mesh = plsc.VectorSubcoreMesh(**_MESH)
    f = pl.kernel(
        _slot_body,
        out_type=jax.ShapeDtypeStruct((NC, NPAD, 128), jnp.float32),
        mesh=mesh,
        compiler_params=pltpu.CompilerParams(needs_layout_passes=False),
        scratch_types=[
            pltpu.VMEM((B,), jnp.int32),
            pltpu.VMEM((B,), jnp.int32),
            pltpu.VMEM((B,), jnp.int32),
            pltpu.VMEM((B,), jnp.int32),
            pltpu.VMEM((B,), jnp.float32),
            pltpu.VMEM((B, 128), jnp.float32),
            pltpu.SemaphoreType.DMA,
            pltpu.VMEM((B,), jnp.int32),
            pltpu.VMEM((B,), jnp.int32),
            pltpu.VMEM((B,), jnp.int32),
            pltpu.VMEM((B,), jnp.int32),
            pltpu.VMEM((B,), jnp.float32),
            pltpu.VMEM((B, 128), jnp.float32),
            pltpu.SemaphoreType.DMA,
            pltpu.VMEM_SHARED((NPAD, 128), jnp.float32),
        ],
    )
    return f(xr, esrc, etype, dst, c)


def _mm2_body(x_ref, w_ref, wr_ref, b_ref, tab_ref, root_ref):
    x = x_ref[...]
    tab_ref[...] = jnp.dot(x, w_ref[...], preferred_element_type=jnp.float32)
    root_ref[...] = (jnp.dot(x, wr_ref[...], preferred_element_type=jnp.float32)
                     + b_ref[0])


def _tc_matmul2(x, wcat, wroot, b, bn):
    n, din = x.shape
    dout = wroot.shape[1]
    return pl.pallas_call(
        _mm2_body,
        grid=(n // bn,),
        in_specs=[
            pl.BlockSpec((bn, din), lambda i: (i, 0)),
            pl.BlockSpec((din, 128), lambda i: (0, 0)),
            pl.BlockSpec((din, dout), lambda i: (0, 0)),
            pl.BlockSpec((1, dout), lambda i: (0, 0)),
        ],
        out_specs=[
            pl.BlockSpec((bn, 128), lambda i: (i, 0)),
            pl.BlockSpec((bn, dout), lambda i: (i, 0)),
        ],
        out_shape=[
            jax.ShapeDtypeStruct((n, 128), jnp.float32),
            jax.ShapeDtypeStruct((n, dout), jnp.float32),
        ],
    )(x, wcat, wroot, b.reshape(1, dout))


def _mm_body(x_ref, w_ref, b_ref, xr_ref, root_ref):
    r = pl.program_id(1)
    acc = jnp.dot(x_ref[...], w_ref[0], preferred_element_type=jnp.float32)

    @pl.when(r < N_REL)
    def _():
        xr_ref[...] = acc

    @pl.when(r == N_REL)
    def _():
        root_ref[...] = acc + b_ref[0]


def _tc_matmul(x, w_all, b, dout, bn):
    n = x.shape[0]
    nb = n // bn
    return pl.pallas_call(
        _mm_body,
        grid=(nb, N_REL + 1),
        in_specs=[
            pl.BlockSpec((bn, x.shape[1]), lambda i, r: (i, 0)),
            pl.BlockSpec((1, x.shape[1], dout), lambda i, r: (r, 0, 0)),
            pl.BlockSpec((1, dout), lambda i, r: (0, 0)),
        ],
        out_specs=[
            pl.BlockSpec((bn, dout),
                         lambda i, r: (jnp.minimum(r, N_REL - 1) * (n // bn) + i, 0)),
            pl.BlockSpec((bn, dout), lambda i, r: (i, 0)),
        ],
        out_shape=[
            jax.ShapeDtypeStruct((N_REL * n, dout), jnp.float32),
            jax.ShapeDtypeStruct((n, dout), jnp.float32),
        ],
    )(x, w_all, b.reshape(1, dout))


def _comb_body(relu, slots, root_ref, p0_ref, p1_ref, o_ref):
    p = p0_ref[0] + p1_ref[0]
    if slots:
        bn = p.shape[0]
        p = p.reshape(bn, 8, p.shape[1] // 8).sum(axis=1)
    v = root_ref[...] + p
    if relu:
        v = jnp.maximum(v, 0.0)
    o_ref[...] = v


def _tc_combine(root, part, relu, bn, slots=False):
    n, d = root.shape
    pd = part.shape[2]
    return pl.pallas_call(
        functools.partial(_comb_body, relu, slots),
        grid=(n // bn,),
        in_specs=[
            pl.BlockSpec((bn, d), lambda i: (i, 0)),
            pl.BlockSpec((1, bn, pd), lambda i: (0, i, 0)),
            pl.BlockSpec((1, bn, pd), lambda i: (1, i, 0)),
        ],
        out_specs=pl.BlockSpec((bn, d), lambda i: (i, 0)),
        out_shape=jax.ShapeDtypeStruct((n, d), jnp.float32),
    )(root, part, part)


def kernel(x, edge_index, edge_type, W_rel1, W_root1, b1, W_rel2, W_root2, b2):
    src = edge_index[0]
    dst = edge_index[1]

    part_cnt, skey, gkey = _sc_count(edge_type, src, dst)
    c = _sc_scale(part_cnt, skey)

    w_all1 = jnp.concatenate([W_rel1, W_root1[None]], axis=0)
    xr1, root1 = _tc_matmul(x, w_all1, b1, 128, 1000)
    part1 = _sc_agg(xr1, gkey, dst, c, D=128)
    h = _tc_combine(root1, part1, relu=True, bn=1000)

    w2cat = jnp.transpose(W_rel2, (1, 0, 2)).reshape(CH1, N_REL * 16)
    tab2, root2 = _tc_matmul2(h, w2cat, W_root2, b2, 1000)
    part2 = _sc_slot_agg(tab2, src, edge_type, dst, c)
    logits = _tc_combine(root2, part2, relu=False, bn=1000, slots=True)
    return logits


# speedup vs baseline: 1.2534x; 1.0005x over previous
import functools

import jax
import jax.numpy as jnp
from jax import lax
from jax.experimental import pallas as pl
from jax.experimental.pallas import tpu as pltpu
from jax.experimental.pallas import tpu_sc as plsc

N_NODES = 10000
N_REL = 8
N_EDGES = 320000
CH1 = 128

NC = 2
NS = 16
NW = NC * NS
LANES = 16

ET = N_EDGES // NW
KPAD = 81920
NPAD = 10240
ROWS_PER_TILE = NPAD // NS

_MESH = dict(core_axis_name="c", subcore_axis_name="s")


def _wid():
    return lax.axis_index("s") * NC + lax.axis_index("c")


def _fill(ref, n, value, dtype):
    vec = jnp.full((LANES,), value, dtype)

    def body(i, _):
        ref[pl.ds(i * LANES, LANES)] = vec
        return 0

    lax.fori_loop(0, n // LANES, body, 0)


def _count_body(etype, src, dst, part_out, skey_out, gkey_out,
                tbuf, sbuf, dbuf, kbuf, gbuf, ones, zbuf, cnt_sh):
    wid = _wid()
    sid = lax.axis_index("s")
    cid = lax.axis_index("c")

    CB = 128
    NCHUNK = N_EDGES // CB
    NITER = -(-NCHUNK // NW)

    _fill(ones, CB, 1.0, jnp.float32)
    _fill(zbuf, KPAD // NS, 0.0, jnp.float32)
    pltpu.sync_copy(zbuf, cnt_sh.at[pl.ds(sid * (KPAD // NS), KPAD // NS)])
    plsc.subcore_barrier()

    def chunk(i, _):
        cidx = i * NW + wid

        @pl.when(cidx < NCHUNK)
        def _():
            base = cidx * CB
            pltpu.sync_copy(etype.at[pl.ds(base, CB)], tbuf)
            pltpu.sync_copy(src.at[pl.ds(base, CB)], sbuf)
            pltpu.sync_copy(dst.at[pl.ds(base, CB)], dbuf)

            def body(j, _):
                sl = pl.ds(j * LANES, LANES)
                t = tbuf[sl] * N_NODES
                kbuf[sl] = t + dbuf[sl]
                gbuf[sl] = t + sbuf[sl]
                return 0

            lax.fori_loop(0, CB // LANES, body, 0)
            pltpu.sync_copy(kbuf, skey_out.at[pl.ds(base, CB)])
            pltpu.sync_copy(gbuf, gkey_out.at[pl.ds(base, CB)])
            pltpu.sync_copy(ones, cnt_sh.at[kbuf], add=True)

        return 0

    lax.fori_loop(0, NITER, chunk, 0)

    plsc.subcore_barrier()
    off = sid * (KPAD // NS)
    pltpu.sync_copy(cnt_sh.at[pl.ds(off, KPAD // NS)],
                    part_out.at[cid, pl.ds(off, KPAD // NS)])


def _sc_count(etype, src, dst):
    mesh = plsc.VectorSubcoreMesh(**_MESH)
    f = pl.kernel(
        _count_body,
        out_type=(
            jax.ShapeDtypeStruct((NC, KPAD), jnp.float32),
            jax.ShapeDtypeStruct((N_EDGES,), jnp.int32),
            jax.ShapeDtypeStruct((N_EDGES,), jnp.int32),
        ),
        mesh=mesh,
        compiler_params=pltpu.CompilerParams(needs_layout_passes=False),
        scratch_types=[
            pltpu.VMEM((128,), jnp.int32),
            pltpu.VMEM((128,), jnp.int32),
            pltpu.VMEM((128,), jnp.int32),
            pltpu.VMEM((128,), jnp.int32),
            pltpu.VMEM((128,), jnp.int32),
            pltpu.VMEM((128,), jnp.float32),
            pltpu.VMEM((KPAD // NS,), jnp.float32),
            pltpu.VMEM_SHARED((KPAD,), jnp.float32),
        ],
    )
    return f(etype, src, dst)


def _scale_body(part, skey, c_out, abuf, bbuf, kbuf, cbuf, tab, cnt_sh):
    sid = lax.axis_index("s")
    wid = _wid()

    W = KPAD // NS
    off = sid * W
    pltpu.sync_copy(part.at[0, pl.ds(off, W)], abuf)
    pltpu.sync_copy(part.at[1, pl.ds(off, W)], bbuf)

    def body(i, _):
        sl = pl.ds(i * LANES, LANES)
        abuf[sl] = abuf[sl] + bbuf[sl]
        return 0

    lax.fori_loop(0, W // LANES, body, 0)
    pltpu.sync_copy(abuf, cnt_sh.at[pl.ds(off, W)])
    plsc.subcore_barrier()

    pltpu.sync_copy(cnt_sh, tab)

    CB = 2000
    one = jnp.full((LANES,), 1.0, jnp.float32)

    for k in range(ET // CB):
        base = wid * ET + k * CB
        pltpu.sync_copy(skey.at[pl.ds(base, CB)], kbuf)

        def body(i, _):
            sl = pl.ds(i * LANES, LANES)
            cnt = plsc.load_gather(tab, [kbuf[sl]])
            cbuf[sl] = one / jnp.maximum(cnt, one)
            return 0

        lax.fori_loop(0, CB // LANES, body, 0)
        pltpu.sync_copy(cbuf, c_out.at[pl.ds(base, CB)])


def _sc_scale(part, skey):
    mesh = plsc.VectorSubcoreMesh(**_MESH)
    W = KPAD // NS
    f = pl.kernel(
        _scale_body,
        out_type=jax.ShapeDtypeStruct((N_EDGES,), jnp.float32),
        mesh=mesh,
        compiler_params=pltpu.CompilerParams(needs_layout_passes=False),
        scratch_types=[
            pltpu.VMEM((W,), jnp.float32),
            pltpu.VMEM((W,), jnp.float32),
            pltpu.VMEM((2000,), jnp.int32),
            pltpu.VMEM((2000,), jnp.float32),
            pltpu.VMEM((KPAD,), jnp.float32),
            pltpu.VMEM_SHARED((KPAD,), jnp.float32),
        ],
    )
    return f(part, skey)


def _agg_body(D, xr, gkey, dst, c, part_out,
              gbuf, dbuf, cbuf, rows, sem, ssem,
              gbuf2, dbuf2, cbuf2, rows2, sem2, ssem2, acc_sh):
    wid = _wid()
    sid = lax.axis_index("s")
    cid = lax.axis_index("c")
    DV = D // LANES
    B = 128
    NCHUNK = N_EDGES // B
    NITER = -(-NCHUNK // NW)

    def zbody(e, _):
        for j in range(DV):
            rows[e, pl.ds(j * LANES, LANES)] = jnp.zeros((LANES,), jnp.float32)
        return 0

    lax.fori_loop(0, B, zbody, 0)
    for z in range(ROWS_PER_TILE // B):
        pltpu.sync_copy(rows,
                        acc_sh.at[pl.ds(sid * ROWS_PER_TILE + z * B, B), :])
    plsc.subcore_barrier()

    bufs = ((gbuf, dbuf, cbuf, rows, sem, ssem),
            (gbuf2, dbuf2, cbuf2, rows2, sem2, ssem2))

    def load_fire(j, bi):
        g, d, cc, r, sm, ss = bufs[bi]
        cidx = j * NW + wid

        if not (isinstance(j, int) and j < 2):
            @pl.when(jnp.logical_and(j >= 2, (j - 2) * NW + wid < NCHUNK))
            def _():
                pltpu.make_async_copy(r, acc_sh.at[d], ss).wait()

        @pl.when(cidx < NCHUNK)
        def _():
            base = cidx * B
            pltpu.sync_copy(gkey.at[pl.ds(base, B)], g)
            pltpu.async_copy(xr.at[g], r, sm)
            pltpu.sync_copy(c.at[pl.ds(base, B)], cc)
            pltpu.sync_copy(dst.at[pl.ds(base, B)], d)

    def drain_process(j, bi):
        g, d, cc, r, sm, ss = bufs[bi]
        cidx = j * NW + wid

        @pl.when(cidx < NCHUNK)
        def _():
            pltpu.make_async_copy(xr.at[g], r, sm).wait()

            def body(e, _):
                cv = plsc.load_gather(cc, [jnp.full((LANES,), e, jnp.int32)])
                for jj in range(DV):
                    sl = pl.ds(jj * LANES, LANES)
                    r[e, sl] = r[e, sl] * cv
                return 0

            lax.fori_loop(0, B, body, 0)
            pltpu.async_copy(r, acc_sh.at[d], ss, add=True)

    load_fire(0, 0)

    def pair(i, _):
        load_fire(2 * i + 1, 1)
        drain_process(2 * i, 0)
        load_fire(2 * i + 2, 0)
        drain_process(2 * i + 1, 1)
        return 0

    lax.fori_loop(0, (NITER + 1) // 2, pair, 0)

    plsc.subcore_barrier()
    roff = sid * ROWS_PER_TILE
    for z in range(ROWS_PER_TILE // B):
        pltpu.sync_copy(acc_sh.at[pl.ds(roff + z * B, B), :],
                        part_out.at[cid, pl.ds(roff + z * B, B), :])


def _sc_agg(xr, gkey, dst, c, D):
    B = 128
    mesh = plsc.VectorSubcoreMesh(**_MESH)
    f = pl.kernel(
        functools.partial(_agg_body, D),
        out_type=jax.ShapeDtypeStruct((NC, NPAD, D), jnp.float32),
        mesh=mesh,
        compiler_params=pltpu.CompilerParams(needs_layout_passes=False),
        scratch_types=[
            pltpu.VMEM((B,), jnp.int32),
            pltpu.VMEM((B,), jnp.int32),
            pltpu.VMEM((B,), jnp.float32),
            pltpu.VMEM((B, D), jnp.float32),
            pltpu.SemaphoreType.DMA,
            pltpu.SemaphoreType.DMA,
            pltpu.VMEM((B,), jnp.int32),
            pltpu.VMEM((B,), jnp.int32),
            pltpu.VMEM((B,), jnp.float32),
            pltpu.VMEM((B, D), jnp.float32),
            pltpu.SemaphoreType.DMA,
            pltpu.SemaphoreType.DMA,
            pltpu.VMEM_SHARED((NPAD, D), jnp.float32),
        ],
    )
    return f(xr, gkey, dst, c)


def _slot_body(xr, esrc, etype, dst, c, part_out,
               kbuf, gbuf, sbuf, dbuf, cbuf, rows, sem, ssem,
               kbuf2, gbuf2, sbuf2, dbuf2, cbuf2, rows2, sem2, ssem2,
               acc_sh):
    wid = _wid()
    sid = lax.axis_index("s")
    cid = lax.axis_index("c")
    B = 128
    NCHUNK = N_EDGES // B
    NITER = -(-NCHUNK // NW)
    iota = lax.iota(jnp.int32, LANES)
    zvec = jnp.zeros((LANES,), jnp.float32)

    def zbody(e, _):
        for j in range(8):
            rows[e, pl.ds(j * LANES, LANES)] = zvec
        return 0

    lax.fori_loop(0, B, zbody, 0)
    for z in range(ROWS_PER_TILE // B):
        pltpu.sync_copy(rows,
                        acc_sh.at[pl.ds(sid * ROWS_PER_TILE + z * B, B), :])
    plsc.subcore_barrier()

    bufs = ((kbuf, gbuf, sbuf, dbuf, cbuf, rows, sem, ssem),
            (kbuf2, gbuf2, sbuf2, dbuf2, cbuf2, rows2, sem2, ssem2))

    def load_fire(j, bi):
        kb_, gb, sb, db, cb, r, sm, ss = bufs[bi]
        cidx = j * NW + wid

        if not (isinstance(j, int) and j < 2):
            @pl.when(jnp.logical_and(j >= 2, (j - 2) * NW + wid < NCHUNK))
            def _():
                pltpu.make_async_copy(r, acc_sh.at[db], ss).wait()

        @pl.when(cidx < NCHUNK)
        def _():
            base = cidx * B
            pltpu.sync_copy(esrc.at[pl.ds(base, B)], gb)
            pltpu.async_copy(xr.at[gb], r, sm)
            pltpu.sync_copy(etype.at[pl.ds(base, B)], kb_)

            def kb(j2, _):
                sl = pl.ds(j2 * LANES, LANES)
                sb[sl] = kb_[sl] * LANES
                return 0

            lax.fori_loop(0, B // LANES, kb, 0)
            pltpu.sync_copy(c.at[pl.ds(base, B)], cb)
            pltpu.sync_copy(dst.at[pl.ds(base, B)], db)

    def drain_process(j, bi):
        _kb, gb, sb, db, cb, r, sm, ss = bufs[bi]
        cidx = j * NW + wid

        @pl.when(cidx < NCHUNK)
        def _():
            pltpu.make_async_copy(xr.at[gb], r, sm).wait()

            def body(e, _):
                ef = jnp.full((LANES,), e, jnp.int32)
                col = plsc.load_gather(sb, [ef]) + iota
                msg = plsc.load_gather(r, [ef, col])
                cv = plsc.load_gather(cb, [ef])
                for j2 in range(8):
                    r[e, pl.ds(j2 * LANES, LANES)] = zvec
                plsc.store_scatter(r, [ef, col], msg * cv)
                return 0

            lax.fori_loop(0, B, body, 0)
            pltpu.async_copy(r, acc_sh.at[db], ss, add=True)

    load_fire(0, 0)

    def pair(i, _):
        load_fire(2 * i + 1, 1)
        drain_process(2 * i, 0)
        load_fire(2 * i + 2, 0)
        drain_process(2 * i + 1, 1)
        return 0

    lax.fori_loop(0, (NITER + 1) // 2, pair, 0)

    plsc.subcore_barrier()

    roff = sid * ROWS_PER_TILE
    for z in range(ROWS_PER_TILE // B):
        pltpu.sync_copy(acc_sh.at[pl.ds(roff + z * B, B), :],
                        part_out.at[cid, pl.ds(roff + z * B, B), :])


def _sc_slot_agg(xr, esrc, etype, dst, c):
    B = 128
    mesh = plsc.VectorSubcoreMesh(**_MESH)
    f = pl.kernel(
        _slot_body,
        out_type=jax.ShapeDtypeStruct((NC, NPAD, 128), jnp.float32),
        mesh=mesh,
        compiler_params=pltpu.CompilerParams(needs_layout_passes=False),
        scratch_types=[
            pltpu.VMEM((B,), jnp.int32),
            pltpu.VMEM((B,), jnp.int32),
            pltpu.VMEM((B,), jnp.int32),
            pltpu.VMEM((B,), jnp.int32),
            pltpu.VMEM((B,), jnp.float32),
            pltpu.VMEM((B, 128), jnp.float32),
            pltpu.SemaphoreType.DMA,
            pltpu.SemaphoreType.DMA,
            pltpu.VMEM((B,), jnp.int32),
            pltpu.VMEM((B,), jnp.int32),
            pltpu.VMEM((B,), jnp.int32),
            pltpu.VMEM((B,), jnp.int32),
            pltpu.VMEM((B,), jnp.float32),
            pltpu.VMEM((B, 128), jnp.float32),
            pltpu.SemaphoreType.DMA,
            pltpu.SemaphoreType.DMA,
            pltpu.VMEM_SHARED((NPAD, 128), jnp.float32),
        ],
    )
    return f(xr, esrc, etype, dst, c)


def _mm2_body(x_ref, w_ref, wr_ref, b_ref, tab_ref, root_ref):
    x = x_ref[...]
    tab_ref[...] = jnp.dot(x, w_ref[...], preferred_element_type=jnp.float32)
    root_ref[...] = (jnp.dot(x, wr_ref[...], preferred_element_type=jnp.float32)
                     + b_ref[0])


def _tc_matmul2(x, wcat, wroot, b, bn):
    n, din = x.shape
    dout = wroot.shape[1]
    return pl.pallas_call(
        _mm2_body,
        grid=(n // bn,),
        in_specs=[
            pl.BlockSpec((bn, din), lambda i: (i, 0)),
            pl.BlockSpec((din, 128), lambda i: (0, 0)),
            pl.BlockSpec((din, dout), lambda i: (0, 0)),
            pl.BlockSpec((1, dout), lambda i: (0, 0)),
        ],
        out_specs=[
            pl.BlockSpec((bn, 128), lambda i: (i, 0)),
            pl.BlockSpec((bn, dout), lambda i: (i, 0)),
        ],
        out_shape=[
            jax.ShapeDtypeStruct((n, 128), jnp.float32),
            jax.ShapeDtypeStruct((n, dout), jnp.float32),
        ],
    )(x, wcat, wroot, b.reshape(1, dout))


def _mm_body(x_ref, w_ref, b_ref, xr_ref, root_ref):
    r = pl.program_id(1)
    acc = jnp.dot(x_ref[...], w_ref[0], preferred_element_type=jnp.float32)

    @pl.when(r < N_REL)
    def _():
        xr_ref[...] = acc

    @pl.when(r == N_REL)
    def _():
        root_ref[...] = acc + b_ref[0]


def _tc_matmul(x, w_all, b, dout, bn):
    n = x.shape[0]
    nb = n // bn
    return pl.pallas_call(
        _mm_body,
        grid=(nb, N_REL + 1),
        in_specs=[
            pl.BlockSpec((bn, x.shape[1]), lambda i, r: (i, 0)),
            pl.BlockSpec((1, x.shape[1], dout), lambda i, r: (r, 0, 0)),
            pl.BlockSpec((1, dout), lambda i, r: (0, 0)),
        ],
        out_specs=[
            pl.BlockSpec((bn, dout),
                         lambda i, r: (jnp.minimum(r, N_REL - 1) * (n // bn) + i, 0)),
            pl.BlockSpec((bn, dout), lambda i, r: (i, 0)),
        ],
        out_shape=[
            jax.ShapeDtypeStruct((N_REL * n, dout), jnp.float32),
            jax.ShapeDtypeStruct((n, dout), jnp.float32),
        ],
    )(x, w_all, b.reshape(1, dout))


def _comb_body(relu, slots, root_ref, p0_ref, p1_ref, o_ref):
    p = p0_ref[0] + p1_ref[0]
    if slots:
        bn = p.shape[0]
        p = p.reshape(bn, 8, p.shape[1] // 8).sum(axis=1)
    v = root_ref[...] + p
    if relu:
        v = jnp.maximum(v, 0.0)
    o_ref[...] = v


def _tc_combine(root, part, relu, bn, slots=False):
    n, d = root.shape
    pd = part.shape[2]
    return pl.pallas_call(
        functools.partial(_comb_body, relu, slots),
        grid=(n // bn,),
        in_specs=[
            pl.BlockSpec((bn, d), lambda i: (i, 0)),
            pl.BlockSpec((1, bn, pd), lambda i: (0, i, 0)),
            pl.BlockSpec((1, bn, pd), lambda i: (1, i, 0)),
        ],
        out_specs=pl.BlockSpec((bn, d), lambda i: (i, 0)),
        out_shape=jax.ShapeDtypeStruct((n, d), jnp.float32),
    )(root, part, part)


def kernel(x, edge_index, edge_type, W_rel1, W_root1, b1, W_rel2, W_root2, b2):
    src = edge_index[0]
    dst = edge_index[1]

    part_cnt, skey, gkey = _sc_count(edge_type, src, dst)
    c = _sc_scale(part_cnt, skey)

    w_all1 = jnp.concatenate([W_rel1, W_root1[None]], axis=0)
    xr1, root1 = _tc_matmul(x, w_all1, b1, 128, 1000)
    part1 = _sc_agg(xr1, gkey, dst, c, D=128)
    h = _tc_combine(root1, part1, relu=True, bn=1000)

    w2cat = jnp.transpose(W_rel2, (1, 0, 2)).reshape(CH1, N_REL * 16)
    tab2, root2 = _tc_matmul2(h, w2cat, W_root2, b2, 1000)
    part2 = _sc_slot_agg(tab2, src, edge_type, dst, c)
    logits = _tc_combine(root2, part2, relu=False, bn=1000, slots=True)
    return logits


# speedup vs baseline: 1.9141x; 1.5271x over previous
import functools

import jax
import jax.numpy as jnp
from jax import lax
from jax.experimental import pallas as pl
from jax.experimental.pallas import tpu as pltpu
from jax.experimental.pallas import tpu_sc as plsc

N_NODES = 10000
N_REL = 8
N_EDGES = 320000
CH1 = 128

NC = 2
NS = 16
NW = NC * NS
LANES = 16

B = 128
NROW = N_EDGES // B
RPAD = 2504
SUP = 8
NSLOT = 80
KPAD = 81920
NPAD = 10240
ROWS_PER_TILE = NPAD // NS

_MESH = dict(core_axis_name="c", subcore_axis_name="s")


def _wid():
    return lax.axis_index("s") * NC + lax.axis_index("c")


def _tile_range(wid):
    start = jnp.where(wid < 25, 80 * wid, 2000 + 72 * (wid - 25))
    cnt = jnp.minimum(jnp.where(wid < 25, 80, 72), NROW - start)
    return start, cnt


def _fill(ref, n, value, dtype):
    vec = jnp.full((LANES,), value, dtype)

    def body(i, _):
        ref[pl.ds(i * LANES, LANES)] = vec
        return 0

    lax.fori_loop(0, n // LANES, body, 0)


def _count_body(etype2, src2, dst2, part_out, skey_out, gkey_out,
                tbigA, sbigA, dbigA, tbigB, sbigB, dbigB,
                kbig, gbig, ones, zbuf, cnt_sh):
    wid = _wid()
    sid = lax.axis_index("s")
    cid = lax.axis_index("c")
    start, cnt = _tile_range(wid)

    _fill(ones, B, 1.0, jnp.float32)
    _fill(zbuf, KPAD // NS, 0.0, jnp.float32)
    pltpu.sync_copy(zbuf, cnt_sh.at[pl.ds(sid * (KPAD // NS), KPAD // NS)])
    plsc.subcore_barrier()

    bigs = ((tbigA, sbigA, dbigA), (tbigB, sbigB, dbigB))

    def load_big(s):
        t, sr, d = bigs[s % 2]

        @pl.when(cnt > SUP * s)
        def _():
            row0 = start + SUP * s
            pltpu.sync_copy(etype2.at[pl.ds(row0, SUP), :], t)
            pltpu.sync_copy(src2.at[pl.ds(row0, SUP), :], sr)
            pltpu.sync_copy(dst2.at[pl.ds(row0, SUP), :], d)

    load_big(0)
    for s in range(NSLOT // SUP):
        t, sr, d = bigs[s % 2]
        if s + 1 < NSLOT // SUP:
            load_big(s + 1)

        @pl.when(cnt > SUP * s)
        def _(s=s, t=t, sr=sr, d=d):
            for k in range(SUP):
                @pl.when(cnt > SUP * s + k)
                def _(k=k):
                    def body(j, _):
                        sl = pl.ds(j * LANES, LANES)
                        tt = t[k, sl] * N_NODES
                        kbig[k, sl] = tt + d[k, sl]
                        gbig[k, sl] = tt + sr[k, sl]
                        return 0

                    lax.fori_loop(0, B // LANES, body, 0)
                    pltpu.sync_copy(ones, cnt_sh.at[kbig.at[k]], add=True)

            row0 = start + SUP * s
            pltpu.sync_copy(kbig, skey_out.at[pl.ds(row0, SUP), :])
            pltpu.sync_copy(gbig, gkey_out.at[pl.ds(row0, SUP), :])

    plsc.subcore_barrier()
    off = sid * (KPAD // NS)
    pltpu.sync_copy(cnt_sh.at[pl.ds(off, KPAD // NS)],
                    part_out.at[cid, pl.ds(off, KPAD // NS)])


def _sc_count(etype2, src2, dst2):
    mesh = plsc.VectorSubcoreMesh(**_MESH)
    big = lambda dt: pltpu.VMEM((SUP, B), dt)
    f = pl.kernel(
        _count_body,
        out_type=(
            jax.ShapeDtypeStruct((NC, KPAD), jnp.float32),
            jax.ShapeDtypeStruct((RPAD, B), jnp.int32),
            jax.ShapeDtypeStruct((RPAD, B), jnp.int32),
        ),
        mesh=mesh,
        compiler_params=pltpu.CompilerParams(needs_layout_passes=False),
        scratch_types=[
            big(jnp.int32), big(jnp.int32), big(jnp.int32),
            big(jnp.int32), big(jnp.int32), big(jnp.int32),
            big(jnp.int32), big(jnp.int32),
            pltpu.VMEM((B,), jnp.float32),
            pltpu.VMEM((KPAD // NS,), jnp.float32),
            pltpu.VMEM_SHARED((KPAD,), jnp.float32),
        ],
    )
    return f(etype2, src2, dst2)


def _scale_body(part, skey, c_out, abuf, bbuf, kbuf, cbuf, tab, cnt_sh):
    sid = lax.axis_index("s")
    wid = _wid()

    W = KPAD // NS
    off = sid * W
    pltpu.sync_copy(part.at[0, pl.ds(off, W)], abuf)
    pltpu.sync_copy(part.at[1, pl.ds(off, W)], bbuf)

    def body(i, _):
        sl = pl.ds(i * LANES, LANES)
        abuf[sl] = abuf[sl] + bbuf[sl]
        return 0

    lax.fori_loop(0, W // LANES, body, 0)
    pltpu.sync_copy(abuf, cnt_sh.at[pl.ds(off, W)])
    plsc.subcore_barrier()

    pltpu.sync_copy(cnt_sh, tab)

    ET = N_EDGES // NW
    CB = 2000
    one = jnp.full((LANES,), 1.0, jnp.float32)

    for k in range(ET // CB):
        base = wid * ET + k * CB
        pltpu.sync_copy(skey.at[pl.ds(base, CB)], kbuf)

        def body(i, _):
            sl = pl.ds(i * LANES, LANES)
            cntv = plsc.load_gather(tab, [kbuf[sl]])
            cbuf[sl] = one / jnp.maximum(cntv, one)
            return 0

        lax.fori_loop(0, CB // LANES, body, 0)
        pltpu.sync_copy(cbuf, c_out.at[pl.ds(base, CB)])


def _sc_scale(part, skey):
    mesh = plsc.VectorSubcoreMesh(**_MESH)
    W = KPAD // NS
    f = pl.kernel(
        _scale_body,
        out_type=jax.ShapeDtypeStruct((RPAD * B,), jnp.float32),
        mesh=mesh,
        compiler_params=pltpu.CompilerParams(needs_layout_passes=False),
        scratch_types=[
            pltpu.VMEM((W,), jnp.float32),
            pltpu.VMEM((W,), jnp.float32),
            pltpu.VMEM((2000,), jnp.int32),
            pltpu.VMEM((2000,), jnp.float32),
            pltpu.VMEM((KPAD,), jnp.float32),
            pltpu.VMEM_SHARED((KPAD,), jnp.float32),
        ],
    )
    return f(part, skey)


def _agg_body(slot, xr, idx2, typ2, dst2, c2, part_out,
              rows, sem, ssem, rows2, sem2, ssem2,
              gbigA, tbigA, dbigA, cbigA, gbigB, tbigB, dbigB, cbigB,
              acc_sh):
    wid = _wid()
    sid = lax.axis_index("s")
    cid = lax.axis_index("c")
    start, cnt = _tile_range(wid)
    iota = lax.iota(jnp.int32, LANES)
    zvec = jnp.zeros((LANES,), jnp.float32)

    def zbody(e, _):
        for j in range(8):
            rows[e, pl.ds(j * LANES, LANES)] = zvec
        return 0

    lax.fori_loop(0, B, zbody, 0)
    for z in range(ROWS_PER_TILE // B):
        pltpu.sync_copy(rows,
                        acc_sh.at[pl.ds(sid * ROWS_PER_TILE + z * B, B), :])
    plsc.subcore_barrier()

    bigs = ((gbigA, tbigA, dbigA, cbigA), (gbigB, tbigB, dbigB, cbigB))
    rbufs = ((rows, sem, ssem), (rows2, sem2, ssem2))

    def load_big(s):
        g, t, d, cc = bigs[s % 2]

        @pl.when(cnt > SUP * s)
        def _():
            row0 = start + SUP * s
            pltpu.sync_copy(idx2.at[pl.ds(row0, SUP), :], g)
            if slot:
                pltpu.sync_copy(typ2.at[pl.ds(row0, SUP), :], t)
            pltpu.sync_copy(dst2.at[pl.ds(row0, SUP), :], d)
            pltpu.sync_copy(c2.at[pl.ds(row0, SUP), :], cc)

    def fire(j):
        g, t, d, cc = bigs[(j // SUP) % 2]
        r, sm, ss = rbufs[j % 2]
        if j >= 2:
            dp = bigs[((j - 2) // SUP) % 2][2]

            @pl.when(cnt > j - 2)
            def _():
                pltpu.make_async_copy(
                    r, acc_sh.at[dp.at[(j - 2) % SUP]], ss).wait()

        @pl.when(cnt > j)
        def _():
            pltpu.async_copy(xr.at[g.at[j % SUP]], r, sm)

    def drain(j):
        g, t, d, cc = bigs[(j // SUP) % 2]
        r, sm, ss = rbufs[j % 2]
        k = j % SUP
        kf = jnp.full((LANES,), k, jnp.int32)

        @pl.when(cnt > j)
        def _():
            pltpu.make_async_copy(xr.at[g.at[k]], r, sm).wait()

            if slot:
                def body(e, _):
                    ef = jnp.full((LANES,), e, jnp.int32)
                    col = plsc.load_gather(t, [kf, ef]) * LANES + iota
                    msg = plsc.load_gather(r, [ef, col])
                    cv = plsc.load_gather(cc, [kf, ef])
                    for j2 in range(8):
                        r[e, pl.ds(j2 * LANES, LANES)] = zvec
                    plsc.store_scatter(r, [ef, col], msg * cv)
                    return 0
            else:
                def body(e, _):
                    ef = jnp.full((LANES,), e, jnp.int32)
                    cv = plsc.load_gather(cc, [kf, ef])
                    for j2 in range(8):
                        sl = pl.ds(j2 * LANES, LANES)
                        r[e, sl] = r[e, sl] * cv
                    return 0

            lax.fori_loop(0, B, body, 0)
            pltpu.async_copy(r, acc_sh.at[d.at[k]], ss, add=True)

    load_big(0)
    fire(0)
    for j in range(NSLOT):
        if j % SUP == SUP - 1 and (j // SUP + 1) < NSLOT // SUP:
            load_big(j // SUP + 1)
        if j + 1 < NSLOT:
            fire(j + 1)
        drain(j)

    for jj in (NSLOT - 2, NSLOT - 1):
        @pl.when(cnt > jj)
        def _(jj=jj):
            r, sm, ss = rbufs[jj % 2]
            d = bigs[(jj // SUP) % 2][2]
            pltpu.make_async_copy(r, acc_sh.at[d.at[jj % SUP]], ss).wait()

    plsc.subcore_barrier()
    roff = sid * ROWS_PER_TILE
    for z in range(ROWS_PER_TILE // B):
        pltpu.sync_copy(acc_sh.at[pl.ds(roff + z * B, B), :],
                        part_out.at[cid, pl.ds(roff + z * B, B), :])


def _sc_agg(xr, idx2, typ2, dst2, c2, slot):
    mesh = plsc.VectorSubcoreMesh(**_MESH)
    big = lambda dt: pltpu.VMEM((SUP, B), dt)
    f = pl.kernel(
        functools.partial(_agg_body, slot),
        out_type=jax.ShapeDtypeStruct((NC, NPAD, 128), jnp.float32),
        mesh=mesh,
        compiler_params=pltpu.CompilerParams(needs_layout_passes=False),
        scratch_types=[
            pltpu.VMEM((B, 128), jnp.float32),
            pltpu.SemaphoreType.DMA,
            pltpu.SemaphoreType.DMA,
            pltpu.VMEM((B, 128), jnp.float32),
            pltpu.SemaphoreType.DMA,
            pltpu.SemaphoreType.DMA,
            big(jnp.int32), big(jnp.int32), big(jnp.int32), big(jnp.float32),
            big(jnp.int32), big(jnp.int32), big(jnp.int32), big(jnp.float32),
            pltpu.VMEM_SHARED((NPAD, 128), jnp.float32),
        ],
    )
    return f(xr, idx2, typ2, dst2, c2)


def _mm2_body(x_ref, w_ref, wr_ref, b_ref, tab_ref, root_ref):
    x = x_ref[...]
    tab_ref[...] = jnp.dot(x, w_ref[...], preferred_element_type=jnp.float32)
    root_ref[...] = (jnp.dot(x, wr_ref[...], preferred_element_type=jnp.float32)
                     + b_ref[0])


def _tc_matmul2(x, wcat, wroot, b, bn):
    n, din = x.shape
    dout = wroot.shape[1]
    return pl.pallas_call(
        _mm2_body,
        grid=(n // bn,),
        in_specs=[
            pl.BlockSpec((bn, din), lambda i: (i, 0)),
            pl.BlockSpec((din, 128), lambda i: (0, 0)),
            pl.BlockSpec((din, dout), lambda i: (0, 0)),
            pl.BlockSpec((1, dout), lambda i: (0, 0)),
        ],
        out_specs=[
            pl.BlockSpec((bn, 128), lambda i: (i, 0)),
            pl.BlockSpec((bn, dout), lambda i: (i, 0)),
        ],
        out_shape=[
            jax.ShapeDtypeStruct((n, 128), jnp.float32),
            jax.ShapeDtypeStruct((n, dout), jnp.float32),
        ],
    )(x, wcat, wroot, b.reshape(1, dout))


def _mm_body(x_ref, w_ref, b_ref, xr_ref, root_ref):
    r = pl.program_id(1)
    acc = jnp.dot(x_ref[...], w_ref[0], preferred_element_type=jnp.float32)

    @pl.when(r < N_REL)
    def _():
        xr_ref[...] = acc

    @pl.when(r == N_REL)
    def _():
        root_ref[...] = acc + b_ref[0]


def _tc_matmul(x, w_all, b, dout, bn):
    n = x.shape[0]
    nb = n // bn
    return pl.pallas_call(
        _mm_body,
        grid=(nb, N_REL + 1),
        in_specs=[
            pl.BlockSpec((bn, x.shape[1]), lambda i, r: (i, 0)),
            pl.BlockSpec((1, x.shape[1], dout), lambda i, r: (r, 0, 0)),
            pl.BlockSpec((1, dout), lambda i, r: (0, 0)),
        ],
        out_specs=[
            pl.BlockSpec((bn, dout),
                         lambda i, r: (jnp.minimum(r, N_REL - 1) * (n // bn) + i, 0)),
            pl.BlockSpec((bn, dout), lambda i, r: (i, 0)),
        ],
        out_shape=[
            jax.ShapeDtypeStruct((N_REL * n, dout), jnp.float32),
            jax.ShapeDtypeStruct((n, dout), jnp.float32),
        ],
    )(x, w_all, b.reshape(1, dout))


def _comb_body(relu, slots, root_ref, p0_ref, p1_ref, o_ref):
    p = p0_ref[0] + p1_ref[0]
    if slots:
        bn = p.shape[0]
        p = p.reshape(bn, 8, p.shape[1] // 8).sum(axis=1)
    v = root_ref[...] + p
    if relu:
        v = jnp.maximum(v, 0.0)
    o_ref[...] = v


def _tc_combine(root, part, relu, bn, slots=False):
    n, d = root.shape
    pd = part.shape[2]
    return pl.pallas_call(
        functools.partial(_comb_body, relu, slots),
        grid=(n // bn,),
        in_specs=[
            pl.BlockSpec((bn, d), lambda i: (i, 0)),
            pl.BlockSpec((1, bn, pd), lambda i: (0, i, 0)),
            pl.BlockSpec((1, bn, pd), lambda i: (1, i, 0)),
        ],
        out_specs=pl.BlockSpec((bn, d), lambda i: (i, 0)),
        out_shape=jax.ShapeDtypeStruct((n, d), jnp.float32),
    )(root, part, part)


def _pad2d(a):
    return jnp.pad(a, (0, RPAD * B - N_EDGES)).reshape(RPAD, B)


def kernel(x, edge_index, edge_type, W_rel1, W_root1, b1, W_rel2, W_root2, b2):
    src2 = _pad2d(edge_index[0])
    dst2 = _pad2d(edge_index[1])
    et2 = _pad2d(edge_type)

    part_cnt, skey2, gkey2 = _sc_count(et2, src2, dst2)
    c = _sc_scale(part_cnt, skey2.reshape(-1))
    c2 = c.reshape(RPAD, B)

    w_all1 = jnp.concatenate([W_rel1, W_root1[None]], axis=0)
    xr1, root1 = _tc_matmul(x, w_all1, b1, 128, 1000)
    part1 = _sc_agg(xr1, gkey2, et2, dst2, c2, slot=False)
    h = _tc_combine(root1, part1, relu=True, bn=1000)

    w2cat = jnp.transpose(W_rel2, (1, 0, 2)).reshape(CH1, N_REL * 16)
    tab2, root2 = _tc_matmul2(h, w2cat, W_root2, b2, 1000)
    part2 = _sc_agg(tab2, src2, et2, dst2, c2, slot=True)
    logits = _tc_combine(root2, part2, relu=False, bn=1000, slots=True)
    return logits


# speedup vs baseline: 1.9415x; 1.0143x over previous
import functools

import jax
import jax.numpy as jnp
from jax import lax
from jax.experimental import pallas as pl
from jax.experimental.pallas import tpu as pltpu
from jax.experimental.pallas import tpu_sc as plsc

N_NODES = 10000
N_REL = 8
N_EDGES = 320000
CH1 = 128

NC = 2
NS = 16
NW = NC * NS
LANES = 16

B = 128
NROW = N_EDGES // B
RPAD = 2504
SUP = 8
NSLOT = 80
KPAD = 81920
NPAD = 10240
ROWS_PER_TILE = NPAD // NS

_MESH = dict(core_axis_name="c", subcore_axis_name="s")


def _wid():
    return lax.axis_index("s") * NC + lax.axis_index("c")


def _tile_range(wid):
    start = jnp.where(wid < 25, 80 * wid, 2000 + 72 * (wid - 25))
    cnt = jnp.minimum(jnp.where(wid < 25, 80, 72), NROW - start)
    return start, cnt


def _fill(ref, n, value, dtype):
    vec = jnp.full((LANES,), value, dtype)

    def body(i, _):
        ref[pl.ds(i * LANES, LANES)] = vec
        return 0

    lax.fori_loop(0, n // LANES, body, 0)


def _count_body(etype2, src2, dst2, part_out, skey_out, gkey_out,
                tbigA, sbigA, dbigA, tbigB, sbigB, dbigB,
                kbig, gbig, ones, zbuf, cnt_sh):
    wid = _wid()
    sid = lax.axis_index("s")
    cid = lax.axis_index("c")
    start, cnt = _tile_range(wid)

    _fill(ones, B, 1.0, jnp.float32)
    _fill(zbuf, KPAD // NS, 0.0, jnp.float32)
    pltpu.sync_copy(zbuf, cnt_sh.at[pl.ds(sid * (KPAD // NS), KPAD // NS)])
    plsc.subcore_barrier()

    bigs = ((tbigA, sbigA, dbigA), (tbigB, sbigB, dbigB))

    def load_big(s):
        t, sr, d = bigs[s % 2]

        @pl.when(cnt > SUP * s)
        def _():
            row0 = start + SUP * s
            pltpu.sync_copy(etype2.at[pl.ds(row0, SUP), :], t)
            pltpu.sync_copy(src2.at[pl.ds(row0, SUP), :], sr)
            pltpu.sync_copy(dst2.at[pl.ds(row0, SUP), :], d)

    load_big(0)
    for s in range(NSLOT // SUP):
        t, sr, d = bigs[s % 2]
        if s + 1 < NSLOT // SUP:
            load_big(s + 1)

        @pl.when(cnt > SUP * s)
        def _(s=s, t=t, sr=sr, d=d):
            for k in range(SUP):
                @pl.when(cnt > SUP * s + k)
                def _(k=k):
                    def body(j, _):
                        sl = pl.ds(j * LANES, LANES)
                        tt = t[k, sl] * N_NODES
                        kbig[k, sl] = tt + d[k, sl]
                        gbig[k, sl] = tt + sr[k, sl]
                        return 0

                    lax.fori_loop(0, B // LANES, body, 0)
                    pltpu.sync_copy(ones, cnt_sh.at[kbig.at[k]], add=True)

            row0 = start + SUP * s
            pltpu.sync_copy(kbig, skey_out.at[pl.ds(row0, SUP), :])
            pltpu.sync_copy(gbig, gkey_out.at[pl.ds(row0, SUP), :])

    plsc.subcore_barrier()
    off = sid * (KPAD // NS)
    pltpu.sync_copy(cnt_sh.at[pl.ds(off, KPAD // NS)],
                    part_out.at[cid, pl.ds(off, KPAD // NS)])


def _sc_count(etype2, src2, dst2):
    mesh = plsc.VectorSubcoreMesh(**_MESH)
    big = lambda dt: pltpu.VMEM((SUP, B), dt)
    f = pl.kernel(
        _count_body,
        out_type=(
            jax.ShapeDtypeStruct((NC, KPAD), jnp.float32),
            jax.ShapeDtypeStruct((RPAD, B), jnp.int32),
            jax.ShapeDtypeStruct((RPAD, B), jnp.int32),
        ),
        mesh=mesh,
        compiler_params=pltpu.CompilerParams(needs_layout_passes=False),
        scratch_types=[
            big(jnp.int32), big(jnp.int32), big(jnp.int32),
            big(jnp.int32), big(jnp.int32), big(jnp.int32),
            big(jnp.int32), big(jnp.int32),
            pltpu.VMEM((B,), jnp.float32),
            pltpu.VMEM((KPAD // NS,), jnp.float32),
            pltpu.VMEM_SHARED((KPAD,), jnp.float32),
        ],
    )
    return f(etype2, src2, dst2)


def _scale_body(part, skey, c_out, abuf, bbuf, kbuf, cbuf, tab, cnt_sh):
    sid = lax.axis_index("s")
    wid = _wid()

    W = KPAD // NS
    off = sid * W
    pltpu.sync_copy(part.at[0, pl.ds(off, W)], abuf)
    pltpu.sync_copy(part.at[1, pl.ds(off, W)], bbuf)

    def body(i, _):
        sl = pl.ds(i * LANES, LANES)
        abuf[sl] = abuf[sl] + bbuf[sl]
        return 0

    lax.fori_loop(0, W // LANES, body, 0)
    pltpu.sync_copy(abuf, cnt_sh.at[pl.ds(off, W)])
    plsc.subcore_barrier()

    pltpu.sync_copy(cnt_sh, tab)

    ET = N_EDGES // NW
    CB = 2000
    one = jnp.full((LANES,), 1.0, jnp.float32)

    for k in range(ET // CB):
        base = wid * ET + k * CB
        pltpu.sync_copy(skey.at[pl.ds(base, CB)], kbuf)

        def body(i, _):
            sl = pl.ds(i * LANES, LANES)
            cntv = plsc.load_gather(tab, [kbuf[sl]])
            cbuf[sl] = one / jnp.maximum(cntv, one)
            return 0

        lax.fori_loop(0, CB // LANES, body, 0)
        pltpu.sync_copy(cbuf, c_out.at[pl.ds(base, CB)])


def _sc_scale(part, skey):
    mesh = plsc.VectorSubcoreMesh(**_MESH)
    W = KPAD // NS
    f = pl.kernel(
        _scale_body,
        out_type=jax.ShapeDtypeStruct((RPAD * B,), jnp.float32),
        mesh=mesh,
        compiler_params=pltpu.CompilerParams(needs_layout_passes=False),
        scratch_types=[
            pltpu.VMEM((W,), jnp.float32),
            pltpu.VMEM((W,), jnp.float32),
            pltpu.VMEM((2000,), jnp.int32),
            pltpu.VMEM((2000,), jnp.float32),
            pltpu.VMEM((KPAD,), jnp.float32),
            pltpu.VMEM_SHARED((KPAD,), jnp.float32),
        ],
    )
    return f(part, skey)


def _agg_body(slot, xr, idx2, typ2, dst2, c2, part_out,
              rows, sem, ssem, rows2, sem2, ssem2,
              gbigA, tbigA, dbigA, cbigA, gbigB, tbigB, dbigB, cbigB,
              acc_sh):
    wid = _wid()
    sid = lax.axis_index("s")
    cid = lax.axis_index("c")
    start, cnt = _tile_range(wid)
    iota = lax.iota(jnp.int32, LANES)
    zvec = jnp.zeros((LANES,), jnp.float32)

    def zbody(e, _):
        for j in range(8):
            rows[e, pl.ds(j * LANES, LANES)] = zvec
        return 0

    lax.fori_loop(0, B, zbody, 0)
    for z in range(ROWS_PER_TILE // B):
        pltpu.sync_copy(rows,
                        acc_sh.at[pl.ds(sid * ROWS_PER_TILE + z * B, B), :])
    plsc.subcore_barrier()

    bigs = ((gbigA, tbigA, dbigA, cbigA), (gbigB, tbigB, dbigB, cbigB))
    rbufs = ((rows, sem, ssem), (rows2, sem2, ssem2))

    def load_big(s):
        g, t, d, cc = bigs[s % 2]

        @pl.when(cnt > SUP * s)
        def _():
            row0 = start + SUP * s
            pltpu.sync_copy(idx2.at[pl.ds(row0, SUP), :], g)
            if slot:
                pltpu.sync_copy(typ2.at[pl.ds(row0, SUP), :], t)
            pltpu.sync_copy(dst2.at[pl.ds(row0, SUP), :], d)
            pltpu.sync_copy(c2.at[pl.ds(row0, SUP), :], cc)

    def fire(j):
        g, t, d, cc = bigs[(j // SUP) % 2]
        r, sm, ss = rbufs[j % 2]
        if j >= 2:
            dp = bigs[((j - 2) // SUP) % 2][2]

            @pl.when(cnt > j - 2)
            def _():
                pltpu.make_async_copy(
                    r, acc_sh.at[dp.at[(j - 2) % SUP]], ss).wait()

        @pl.when(cnt > j)
        def _():
            pltpu.async_copy(xr.at[g.at[j % SUP]], r, sm)

    def drain(j):
        g, t, d, cc = bigs[(j // SUP) % 2]
        r, sm, ss = rbufs[j % 2]
        k = j % SUP
        kf = jnp.full((LANES,), k, jnp.int32)

        @pl.when(cnt > j)
        def _():
            pltpu.make_async_copy(xr.at[g.at[k]], r, sm).wait()

            if slot:
                def body(e, _):
                    ef = jnp.full((LANES,), e, jnp.int32)
                    col = plsc.load_gather(t, [kf, ef]) * LANES + iota
                    msg = plsc.load_gather(r, [ef, col])
                    cv = plsc.load_gather(cc, [kf, ef])
                    for j2 in range(8):
                        r[e, pl.ds(j2 * LANES, LANES)] = zvec
                    plsc.store_scatter(r, [ef, col], msg * cv)
                    return 0
            else:
                def body(e, _):
                    ef = jnp.full((LANES,), e, jnp.int32)
                    cv = plsc.load_gather(cc, [kf, ef])
                    for j2 in range(8):
                        sl = pl.ds(j2 * LANES, LANES)
                        r[e, sl] = r[e, sl] * cv
                    return 0

            lax.fori_loop(0, B, body, 0)
            pltpu.async_copy(r, acc_sh.at[d.at[k]], ss, add=True)

    load_big(0)
    fire(0)
    for j in range(NSLOT):
        if j % SUP == SUP - 1 and (j // SUP + 1) < NSLOT // SUP:
            load_big(j // SUP + 1)
        if j + 1 < NSLOT:
            fire(j + 1)
        drain(j)

    for jj in (NSLOT - 2, NSLOT - 1):
        @pl.when(cnt > jj)
        def _(jj=jj):
            r, sm, ss = rbufs[jj % 2]
            d = bigs[(jj // SUP) % 2][2]
            pltpu.make_async_copy(r, acc_sh.at[d.at[jj % SUP]], ss).wait()

    plsc.subcore_barrier()
    roff = sid * ROWS_PER_TILE
    for z in range(ROWS_PER_TILE // B):
        pltpu.sync_copy(acc_sh.at[pl.ds(roff + z * B, B), :],
                        part_out.at[cid, pl.ds(roff + z * B, B), :])


def _sc_agg(xr, idx2, typ2, dst2, c2, slot):
    mesh = plsc.VectorSubcoreMesh(**_MESH)
    big = lambda dt: pltpu.VMEM((SUP, B), dt)
    f = pl.kernel(
        functools.partial(_agg_body, slot),
        out_type=jax.ShapeDtypeStruct((NC, NPAD, 128), jnp.float32),
        mesh=mesh,
        compiler_params=pltpu.CompilerParams(needs_layout_passes=False),
        scratch_types=[
            pltpu.VMEM((B, 128), jnp.float32),
            pltpu.SemaphoreType.DMA,
            pltpu.SemaphoreType.DMA,
            pltpu.VMEM((B, 128), jnp.float32),
            pltpu.SemaphoreType.DMA,
            pltpu.SemaphoreType.DMA,
            big(jnp.int32), big(jnp.int32), big(jnp.int32), big(jnp.float32),
            big(jnp.int32), big(jnp.int32), big(jnp.int32), big(jnp.float32),
            pltpu.VMEM_SHARED((NPAD, 128), jnp.float32),
        ],
    )
    return f(xr, idx2, typ2, dst2, c2)


def _mm2_body(root_ref, p0_ref, p1_ref, w_ref, wr_ref, b_ref,
              tab_ref, root_ref2):
    h = jnp.maximum(root_ref[...] + p0_ref[0] + p1_ref[0], 0.0)
    tab_ref[...] = jnp.dot(h, w_ref[...], preferred_element_type=jnp.float32)
    root_ref2[...] = (jnp.dot(h, wr_ref[...], preferred_element_type=jnp.float32)
                      + b_ref[0])


def _tc_matmul2(root1, part1, wcat, wroot, b, bn):
    n, din = root1.shape
    dout = wroot.shape[1]
    return pl.pallas_call(
        _mm2_body,
        grid=(n // bn,),
        in_specs=[
            pl.BlockSpec((bn, din), lambda i: (i, 0)),
            pl.BlockSpec((1, bn, din), lambda i: (0, i, 0)),
            pl.BlockSpec((1, bn, din), lambda i: (1, i, 0)),
            pl.BlockSpec((din, 128), lambda i: (0, 0)),
            pl.BlockSpec((din, dout), lambda i: (0, 0)),
            pl.BlockSpec((1, dout), lambda i: (0, 0)),
        ],
        out_specs=[
            pl.BlockSpec((bn, 128), lambda i: (i, 0)),
            pl.BlockSpec((bn, dout), lambda i: (i, 0)),
        ],
        out_shape=[
            jax.ShapeDtypeStruct((n, 128), jnp.float32),
            jax.ShapeDtypeStruct((n, dout), jnp.float32),
        ],
    )(root1, part1, part1, wcat, wroot, b.reshape(1, dout))


def _mm_body(x_ref, w_ref, b_ref, xr_ref, root_ref):
    r = pl.program_id(1)
    acc = jnp.dot(x_ref[...], w_ref[0], preferred_element_type=jnp.float32)

    @pl.when(r < N_REL)
    def _():
        xr_ref[...] = acc

    @pl.when(r == N_REL)
    def _():
        root_ref[...] = acc + b_ref[0]


def _tc_matmul(x, w_all, b, dout, bn):
    n = x.shape[0]
    nb = n // bn
    return pl.pallas_call(
        _mm_body,
        grid=(nb, N_REL + 1),
        in_specs=[
            pl.BlockSpec((bn, x.shape[1]), lambda i, r: (i, 0)),
            pl.BlockSpec((1, x.shape[1], dout), lambda i, r: (r, 0, 0)),
            pl.BlockSpec((1, dout), lambda i, r: (0, 0)),
        ],
        out_specs=[
            pl.BlockSpec((bn, dout),
                         lambda i, r: (jnp.minimum(r, N_REL - 1) * (n // bn) + i, 0)),
            pl.BlockSpec((bn, dout), lambda i, r: (i, 0)),
        ],
        out_shape=[
            jax.ShapeDtypeStruct((N_REL * n, dout), jnp.float32),
            jax.ShapeDtypeStruct((n, dout), jnp.float32),
        ],
    )(x, w_all, b.reshape(1, dout))


def _comb_body(relu, slots, root_ref, p0_ref, p1_ref, o_ref):
    p = p0_ref[0] + p1_ref[0]
    if slots:
        bn = p.shape[0]
        p = p.reshape(bn, 8, p.shape[1] // 8).sum(axis=1)
    v = root_ref[...] + p
    if relu:
        v = jnp.maximum(v, 0.0)
    o_ref[...] = v


def _tc_combine(root, part, relu, bn, slots=False):
    n, d = root.shape
    pd = part.shape[2]
    return pl.pallas_call(
        functools.partial(_comb_body, relu, slots),
        grid=(n // bn,),
        in_specs=[
            pl.BlockSpec((bn, d), lambda i: (i, 0)),
            pl.BlockSpec((1, bn, pd), lambda i: (0, i, 0)),
            pl.BlockSpec((1, bn, pd), lambda i: (1, i, 0)),
        ],
        out_specs=pl.BlockSpec((bn, d), lambda i: (i, 0)),
        out_shape=jax.ShapeDtypeStruct((n, d), jnp.float32),
    )(root, part, part)


def _pad2d(a):
    return jnp.pad(a, (0, RPAD * B - N_EDGES)).reshape(RPAD, B)


def kernel(x, edge_index, edge_type, W_rel1, W_root1, b1, W_rel2, W_root2, b2):
    src2 = _pad2d(edge_index[0])
    dst2 = _pad2d(edge_index[1])
    et2 = _pad2d(edge_type)

    part_cnt, skey2, gkey2 = _sc_count(et2, src2, dst2)
    c = _sc_scale(part_cnt, skey2.reshape(-1))
    c2 = c.reshape(RPAD, B)

    w_all1 = jnp.concatenate([W_rel1, W_root1[None]], axis=0)
    xr1, root1 = _tc_matmul(x, w_all1, b1, 128, 1000)
    part1 = _sc_agg(xr1, gkey2, et2, dst2, c2, slot=False)

    w2cat = jnp.transpose(W_rel2, (1, 0, 2)).reshape(CH1, N_REL * 16)
    tab2, root2 = _tc_matmul2(root1, part1, w2cat, W_root2, b2, 1000)
    part2 = _sc_agg(tab2, src2, et2, dst2, c2, slot=True)
    logits = _tc_combine(root2, part2, relu=False, bn=1000, slots=True)
    return logits
